# Initial kernel scaffold; baseline (speedup 1.0000x reference)
import jax
import jax.numpy as jnp
from jax.experimental import pallas as pl


def kernel(nf, ef, edge_index, We0, be0, wa0, Wn0, bn0, We1, be1, wa1, Wn1, bn1, We2, be2, wa2, Wn2, bn2, Wnode, bnode, Wedge, bedge):
    raise NotImplementedError("write your pallas kernel here")


# speedup vs baseline: 1.3431x; 1.3431x over previous
import functools
import jax
import jax.numpy as jnp
from jax import lax
from jax.experimental import pallas as pl
from jax.experimental.pallas import tpu as pltpu

N = 10000
E = 320000
H = 128
BE = 2560
NBE = E // BE
BN = 1000
NBN = N // BN


def _t0_body(nf_ref, ws_ref, wd_ref, p_ref, q_ref):
    nf = nf_ref[...]
    p_ref[...] = jnp.dot(nf, ws_ref[...], preferred_element_type=jnp.float32)
    q_ref[...] = jnp.dot(nf, wd_ref[...], preferred_element_type=jnp.float32)


def _tc_pq(nf, Ws, Wd):
    return pl.pallas_call(
        _t0_body,
        grid=(NBN,),
        in_specs=[
            pl.BlockSpec((BN, H), lambda c: (c, 0)),
            pl.BlockSpec((H, H), lambda c: (0, 0)),
            pl.BlockSpec((H, H), lambda c: (0, 0)),
        ],
        out_specs=[
            pl.BlockSpec((BN, H), lambda c: (c, 0)),
            pl.BlockSpec((BN, H), lambda c: (c, 0)),
        ],
        out_shape=[
            jax.ShapeDtypeStruct((N, H), jnp.float32),
            jax.ShapeDtypeStruct((N, H), jnp.float32),
        ],
    )(nf, Ws, Wd)


def _t1_body(ps_ref, qd_ref, ef_ref, we_ref, be_ref, wa_ref, e_ref, lg_ref):
    z = (ps_ref[...] + qd_ref[...]
         + jnp.dot(ef_ref[...], we_ref[...], preferred_element_type=jnp.float32)
         + be_ref[...])
    e = jnp.maximum(z, 0.0)
    e_ref[...] = e
    c = pl.program_id(0)
    lg_ref[pl.ds(c * BE, BE)] = jnp.sum(e * wa_ref[...], axis=1)


def _tc_edge(psrc, qdst, ef, We_ef, be, wa):
    ed = ef.shape[1]
    return pl.pallas_call(
        _t1_body,
        grid=(NBE,),
        in_specs=[
            pl.BlockSpec((BE, H), lambda c: (c, 0)),
            pl.BlockSpec((BE, H), lambda c: (c, 0)),
            pl.BlockSpec((BE, ed), lambda c: (c, 0)),
            pl.BlockSpec((ed, H), lambda c: (0, 0)),
            pl.BlockSpec((H,), lambda c: (0,)),
            pl.BlockSpec((H,), lambda c: (0,)),
        ],
        out_specs=[
            pl.BlockSpec((BE, H), lambda c: (c, 0)),
            pl.BlockSpec((E,), lambda c: (0,)),
        ],
        out_shape=[
            jax.ShapeDtypeStruct((E, H), jnp.float32),
            jax.ShapeDtypeStruct((E,), jnp.float32),
        ],
    )(psrc, qdst, ef, We_ef, be, wa)


def _t3_body(nf_ref, a0_ref, a1_ref, wna_ref, wnb_ref, bn_ref, ws_ref, wd_ref,
             n_ref, p_ref, q_ref):
    agg = a0_ref[...] + a1_ref[...]
    n_new = jnp.maximum(
        jnp.dot(nf_ref[...], wna_ref[...], preferred_element_type=jnp.float32)
        + jnp.dot(agg, wnb_ref[...], preferred_element_type=jnp.float32)
        + bn_ref[...], 0.0)
    n_ref[...] = n_new
    p_ref[...] = jnp.dot(n_new, ws_ref[...], preferred_element_type=jnp.float32)
    q_ref[...] = jnp.dot(n_new, wd_ref[...], preferred_element_type=jnp.float32)


def _tc_node(nf, agg0, agg1, Wn_a, Wn_b, bn, Ws_next, Wd_next):
    nd = nf.shape[1]
    return pl.pallas_call(
        _t3_body,
        grid=(NBN,),
        in_specs=[
            pl.BlockSpec((BN, nd), lambda c: (c, 0)),
            pl.BlockSpec((BN, H), lambda c: (c, 0)),
            pl.BlockSpec((BN, H), lambda c: (c, 0)),
            pl.BlockSpec((nd, H), lambda c: (0, 0)),
            pl.BlockSpec((H, H), lambda c: (0, 0)),
            pl.BlockSpec((H,), lambda c: (0,)),
            pl.BlockSpec((H, H), lambda c: (0, 0)),
            pl.BlockSpec((H, H), lambda c: (0, 0)),
        ],
        out_specs=[
            pl.BlockSpec((BN, H), lambda c: (c, 0)),
            pl.BlockSpec((BN, H), lambda c: (c, 0)),
            pl.BlockSpec((BN, H), lambda c: (c, 0)),
        ],
        out_shape=[
            jax.ShapeDtypeStruct((N, H), jnp.float32),
            jax.ShapeDtypeStruct((N, H), jnp.float32),
            jax.ShapeDtypeStruct((N, H), jnp.float32),
        ],
    )(nf, agg0, agg1, Wn_a, Wn_b, bn, Ws_next, Wd_next)


def _t3f_body(nf_ref, a0_ref, a1_ref, wna_ref, wnb_ref, bn_ref, wo_ref, bo_ref,
              u_ref):
    agg = a0_ref[...] + a1_ref[...]
    n_new = jnp.maximum(
        jnp.dot(nf_ref[...], wna_ref[...], preferred_element_type=jnp.float32)
        + jnp.dot(agg, wnb_ref[...], preferred_element_type=jnp.float32)
        + bn_ref[...], 0.0)
    u_ref[...] = (jnp.dot(n_new, wo_ref[...], preferred_element_type=jnp.float32)
                  + bo_ref[...])


def _tc_node_final(nf, agg0, agg1, Wn_a, Wn_b, bn, Wnode, bnode):
    nout = Wnode.shape[1]
    return pl.pallas_call(
        _t3f_body,
        grid=(NBN,),
        in_specs=[
            pl.BlockSpec((BN, H), lambda c: (c, 0)),
            pl.BlockSpec((BN, H), lambda c: (c, 0)),
            pl.BlockSpec((BN, H), lambda c: (c, 0)),
            pl.BlockSpec((H, H), lambda c: (0, 0)),
            pl.BlockSpec((H, H), lambda c: (0, 0)),
            pl.BlockSpec((H,), lambda c: (0,)),
            pl.BlockSpec((H, nout), lambda c: (0, 0)),
            pl.BlockSpec((nout,), lambda c: (0,)),
        ],
        out_specs=pl.BlockSpec((BN, nout), lambda c: (c, 0)),
        out_shape=jax.ShapeDtypeStruct((N, nout), jnp.float32),
    )(nf, agg0, agg1, Wn_a, Wn_b, bn, Wnode, bnode)


def _t4_body(ef_ref, w_ref, b_ref, u_ref):
    u_ref[...] = (jnp.dot(ef_ref[...], w_ref[...],
                          preferred_element_type=jnp.float32) + b_ref[...])


def _tc_edge_head(ef, Wedge, bedge):
    eout = Wedge.shape[1]
    return pl.pallas_call(
        _t4_body,
        grid=(NBE,),
        in_specs=[
            pl.BlockSpec((BE, H), lambda c: (c, 0)),
            pl.BlockSpec((H, eout), lambda c: (0, 0)),
            pl.BlockSpec((eout,), lambda c: (0,)),
        ],
        out_specs=pl.BlockSpec((BE, eout), lambda c: (c, 0)),
        out_shape=jax.ShapeDtypeStruct((E, eout), jnp.float32),
    )(ef, Wedge, bedge)


def _segment_softmax_agg(e, logit, dst):
    gmax = jnp.max(logit)
    ex = jnp.exp(logit - gmax)
    den = jax.ops.segment_sum(ex, dst, num_segments=N)
    attn = ex / (den[dst] + 1e-16)
    m = e * attn[:, None]
    h = E // 2
    agg0 = jax.ops.segment_sum(m[:h], dst[:h], num_segments=N)
    agg1 = jax.ops.segment_sum(m[h:], dst[h:], num_segments=N)
    return agg0, agg1


def kernel(nf, ef, edge_index, We0, be0, wa0, Wn0, bn0, We1, be1, wa1, Wn1,
           bn1, We2, be2, wa2, Wn2, bn2, Wnode, bnode, Wedge, bedge):
    src = edge_index[0]
    dst = edge_index[1]
    params = [(We0, be0, wa0, Wn0, bn0), (We1, be1, wa1, Wn1, bn1),
              (We2, be2, wa2, Wn2, bn2)]
    nds = [nf.shape[1], H, H]
    splits = []
    for l, (We, be, wa, Wn, bn) in enumerate(params):
        nd = nds[l]
        We_s, We_d, We_e = We[:nd], We[nd:2 * nd], We[2 * nd:]
        Wn_a, Wn_b = Wn[:nd], Wn[nd:]
        splits.append((We_s, We_d, We_e, be, wa, Wn_a, Wn_b, bn))

    P, Q = _tc_pq(nf, splits[0][0], splits[0][1])
    for l in range(3):
        We_s, We_d, We_e, be, wa, Wn_a, Wn_b, bn = splits[l]
        psrc = jnp.take(P, src, axis=0)
        qdst = jnp.take(Q, dst, axis=0)
        e, logit = _tc_edge(psrc, qdst, ef, We_e, be, wa)
        agg0, agg1 = _segment_softmax_agg(e, logit, dst)
        if l < 2:
            P, Q = None, None
            nf, P, Q = _tc_node(nf, agg0, agg1, Wn_a, Wn_b, bn,
                                splits[l + 1][0], splits[l + 1][1])
        else:
            unf = _tc_node_final(nf, agg0, agg1, Wn_a, Wn_b, bn, Wnode, bnode)
        ef = e
    uef = _tc_edge_head(ef, Wedge, bedge)
    return unf, uef


# speedup vs baseline: 1.8603x; 1.3851x over previous
import functools
import jax
import jax.numpy as jnp
from jax import lax
from jax.experimental import pallas as pl
from jax.experimental.pallas import tpu as pltpu
from jax.experimental.pallas import tpu_sc as plsc

N = 10000
E = 320000
H = 128
NW = 32
EPW = E // NW
GB = 80
NGC = EPW // GB
BE = 2560
NBE = E // BE
BN = 1000
NBN = N // BN


def _s1_body(p_hbm, q_hbm, src_hbm, dst_hbm, ps_hbm, qd_hbm,
             src_v, dst_v, bufp, bufq, semp, semq):
    wid = lax.axis_index("s") * 2 + lax.axis_index("c")
    base = wid * EPW
    pltpu.sync_copy(src_hbm.at[pl.ds(base, EPW)], src_v)
    pltpu.sync_copy(dst_hbm.at[pl.ds(base, EPW)], dst_v)

    def chunk(c, carry):
        off = c * GB
        gp = pltpu.async_copy(p_hbm.at[src_v.at[pl.ds(off, GB)]], bufp, semp)
        gq = pltpu.async_copy(q_hbm.at[dst_v.at[pl.ds(off, GB)]], bufq, semq)
        gp.wait()
        gq.wait()
        wp = pltpu.async_copy(bufp, ps_hbm.at[pl.ds(base + off, GB)], semp)
        wq = pltpu.async_copy(bufq, qd_hbm.at[pl.ds(base + off, GB)], semq)
        wp.wait()
        wq.wait()
        return carry

    lax.fori_loop(0, NGC, chunk, 0)


def _sc_gather(P, Q, src, dst):
    f = functools.partial(
        pl.kernel,
        out_type=[
            jax.ShapeDtypeStruct((E, H), jnp.float32),
            jax.ShapeDtypeStruct((E, H), jnp.float32),
        ],
        mesh=plsc.VectorSubcoreMesh(core_axis_name="c", subcore_axis_name="s"),
        scratch_types=[
            pltpu.VMEM((EPW,), jnp.int32),
            pltpu.VMEM((EPW,), jnp.int32),
            pltpu.VMEM((GB, H), jnp.float32),
            pltpu.VMEM((GB, H), jnp.float32),
            pltpu.SemaphoreType.DMA,
            pltpu.SemaphoreType.DMA,
        ],
    )(_s1_body)
    return f(P, Q, src, dst)


def _t0_body(nf_ref, ws_ref, wd_ref, p_ref, q_ref):
    nf = nf_ref[...]
    p_ref[...] = jnp.dot(nf, ws_ref[...], preferred_element_type=jnp.float32)
    q_ref[...] = jnp.dot(nf, wd_ref[...], preferred_element_type=jnp.float32)


def _tc_pq(nf, Ws, Wd):
    return pl.pallas_call(
        _t0_body,
        grid=(NBN,),
        in_specs=[
            pl.BlockSpec((BN, H), lambda c: (c, 0)),
            pl.BlockSpec((H, H), lambda c: (0, 0)),
            pl.BlockSpec((H, H), lambda c: (0, 0)),
        ],
        out_specs=[
            pl.BlockSpec((BN, H), lambda c: (c, 0)),
            pl.BlockSpec((BN, H), lambda c: (c, 0)),
        ],
        out_shape=[
            jax.ShapeDtypeStruct((N, H), jnp.float32),
            jax.ShapeDtypeStruct((N, H), jnp.float32),
        ],
    )(nf, Ws, Wd)


def _t1_body(ps_ref, qd_ref, ef_ref, we_ref, be_ref, wa_ref, e_ref, lg_ref):
    z = (ps_ref[...] + qd_ref[...]
         + jnp.dot(ef_ref[...], we_ref[...], preferred_element_type=jnp.float32)
         + be_ref[...])
    e = jnp.maximum(z, 0.0)
    e_ref[...] = e
    c = pl.program_id(0)
    lg_ref[pl.ds(c * BE, BE)] = jnp.sum(e * wa_ref[...], axis=1)


def _tc_edge(psrc, qdst, ef, We_ef, be, wa):
    ed = ef.shape[1]
    return pl.pallas_call(
        _t1_body,
        grid=(NBE,),
        in_specs=[
            pl.BlockSpec((BE, H), lambda c: (c, 0)),
            pl.BlockSpec((BE, H), lambda c: (c, 0)),
            pl.BlockSpec((BE, ed), lambda c: (c, 0)),
            pl.BlockSpec((ed, H), lambda c: (0, 0)),
            pl.BlockSpec((H,), lambda c: (0,)),
            pl.BlockSpec((H,), lambda c: (0,)),
        ],
        out_specs=[
            pl.BlockSpec((BE, H), lambda c: (c, 0)),
            pl.BlockSpec((E,), lambda c: (0,)),
        ],
        out_shape=[
            jax.ShapeDtypeStruct((E, H), jnp.float32),
            jax.ShapeDtypeStruct((E,), jnp.float32),
        ],
    )(psrc, qdst, ef, We_ef, be, wa)


def _t3_body(nf_ref, a0_ref, a1_ref, wna_ref, wnb_ref, bn_ref, ws_ref, wd_ref,
             n_ref, p_ref, q_ref):
    agg = a0_ref[...] + a1_ref[...]
    n_new = jnp.maximum(
        jnp.dot(nf_ref[...], wna_ref[...], preferred_element_type=jnp.float32)
        + jnp.dot(agg, wnb_ref[...], preferred_element_type=jnp.float32)
        + bn_ref[...], 0.0)
    n_ref[...] = n_new
    p_ref[...] = jnp.dot(n_new, ws_ref[...], preferred_element_type=jnp.float32)
    q_ref[...] = jnp.dot(n_new, wd_ref[...], preferred_element_type=jnp.float32)


def _tc_node(nf, agg0, agg1, Wn_a, Wn_b, bn, Ws_next, Wd_next):
    nd = nf.shape[1]
    return pl.pallas_call(
        _t3_body,
        grid=(NBN,),
        in_specs=[
            pl.BlockSpec((BN, nd), lambda c: (c, 0)),
            pl.BlockSpec((BN, H), lambda c: (c, 0)),
            pl.BlockSpec((BN, H), lambda c: (c, 0)),
            pl.BlockSpec((nd, H), lambda c: (0, 0)),
            pl.BlockSpec((H, H), lambda c: (0, 0)),
            pl.BlockSpec((H,), lambda c: (0,)),
            pl.BlockSpec((H, H), lambda c: (0, 0)),
            pl.BlockSpec((H, H), lambda c: (0, 0)),
        ],
        out_specs=[
            pl.BlockSpec((BN, H), lambda c: (c, 0)),
            pl.BlockSpec((BN, H), lambda c: (c, 0)),
            pl.BlockSpec((BN, H), lambda c: (c, 0)),
        ],
        out_shape=[
            jax.ShapeDtypeStruct((N, H), jnp.float32),
            jax.ShapeDtypeStruct((N, H), jnp.float32),
            jax.ShapeDtypeStruct((N, H), jnp.float32),
        ],
    )(nf, agg0, agg1, Wn_a, Wn_b, bn, Ws_next, Wd_next)


def _t3f_body(nf_ref, a0_ref, a1_ref, wna_ref, wnb_ref, bn_ref, wo_ref, bo_ref,
              u_ref):
    agg = a0_ref[...] + a1_ref[...]
    n_new = jnp.maximum(
        jnp.dot(nf_ref[...], wna_ref[...], preferred_element_type=jnp.float32)
        + jnp.dot(agg, wnb_ref[...], preferred_element_type=jnp.float32)
        + bn_ref[...], 0.0)
    u_ref[...] = (jnp.dot(n_new, wo_ref[...], preferred_element_type=jnp.float32)
                  + bo_ref[...])


def _tc_node_final(nf, agg0, agg1, Wn_a, Wn_b, bn, Wnode, bnode):
    nout = Wnode.shape[1]
    return pl.pallas_call(
        _t3f_body,
        grid=(NBN,),
        in_specs=[
            pl.BlockSpec((BN, H), lambda c: (c, 0)),
            pl.BlockSpec((BN, H), lambda c: (c, 0)),
            pl.BlockSpec((BN, H), lambda c: (c, 0)),
            pl.BlockSpec((H, H), lambda c: (0, 0)),
            pl.BlockSpec((H, H), lambda c: (0, 0)),
            pl.BlockSpec((H,), lambda c: (0,)),
            pl.BlockSpec((H, nout), lambda c: (0, 0)),
            pl.BlockSpec((nout,), lambda c: (0,)),
        ],
        out_specs=pl.BlockSpec((BN, nout), lambda c: (c, 0)),
        out_shape=jax.ShapeDtypeStruct((N, nout), jnp.float32),
    )(nf, agg0, agg1, Wn_a, Wn_b, bn, Wnode, bnode)


def _t4_body(ef_ref, w_ref, b_ref, u_ref):
    u_ref[...] = (jnp.dot(ef_ref[...], w_ref[...],
                          preferred_element_type=jnp.float32) + b_ref[...])


def _tc_edge_head(ef, Wedge, bedge):
    eout = Wedge.shape[1]
    return pl.pallas_call(
        _t4_body,
        grid=(NBE,),
        in_specs=[
            pl.BlockSpec((BE, H), lambda c: (c, 0)),
            pl.BlockSpec((H, eout), lambda c: (0, 0)),
            pl.BlockSpec((eout,), lambda c: (0,)),
        ],
        out_specs=pl.BlockSpec((BE, eout), lambda c: (c, 0)),
        out_shape=jax.ShapeDtypeStruct((E, eout), jnp.float32),
    )(ef, Wedge, bedge)


def _segment_softmax_agg(e, logit, dst):
    gmax = jnp.max(logit)
    ex = jnp.exp(logit - gmax)
    den = jax.ops.segment_sum(ex, dst, num_segments=N)
    attn = ex / (den[dst] + 1e-16)
    m = e * attn[:, None]
    h = E // 2
    agg0 = jax.ops.segment_sum(m[:h], dst[:h], num_segments=N)
    agg1 = jax.ops.segment_sum(m[h:], dst[h:], num_segments=N)
    return agg0, agg1


def kernel(nf, ef, edge_index, We0, be0, wa0, Wn0, bn0, We1, be1, wa1, Wn1,
           bn1, We2, be2, wa2, Wn2, bn2, Wnode, bnode, Wedge, bedge):
    src = edge_index[0]
    dst = edge_index[1]
    params = [(We0, be0, wa0, Wn0, bn0), (We1, be1, wa1, Wn1, bn1),
              (We2, be2, wa2, Wn2, bn2)]
    nds = [nf.shape[1], H, H]
    splits = []
    for l, (We, be, wa, Wn, bn) in enumerate(params):
        nd = nds[l]
        We_s, We_d, We_e = We[:nd], We[nd:2 * nd], We[2 * nd:]
        Wn_a, Wn_b = Wn[:nd], Wn[nd:]
        splits.append((We_s, We_d, We_e, be, wa, Wn_a, Wn_b, bn))

    P, Q = _tc_pq(nf, splits[0][0], splits[0][1])
    for l in range(3):
        We_s, We_d, We_e, be, wa, Wn_a, Wn_b, bn = splits[l]
        psrc, qdst = _sc_gather(P, Q, src, dst)
        e, logit = _tc_edge(psrc, qdst, ef, We_e, be, wa)
        agg0, agg1 = _segment_softmax_agg(e, logit, dst)
        if l < 2:
            P, Q = None, None
            nf, P, Q = _tc_node(nf, agg0, agg1, Wn_a, Wn_b, bn,
                                splits[l + 1][0], splits[l + 1][1])
        else:
            unf = _tc_node_final(nf, agg0, agg1, Wn_a, Wn_b, bn, Wnode, bnode)
        ef = e
    uef = _tc_edge_head(ef, Wedge, bedge)
    return unf, uef


# speedup vs baseline: 5.1993x; 2.7948x over previous
import functools
import jax
import jax.numpy as jnp
from jax import lax
from jax.experimental import pallas as pl
from jax.experimental.pallas import tpu as pltpu
from jax.experimental.pallas import tpu_sc as plsc

N = 10000
E = 320000
H = 128
NW = 32
EPW = E // NW
GB = 80
NGC = EPW // GB
BE = 2560
NBE = E // BE
BN = 1000
NBN = N // BN


def _s1_body(p_hbm, q_hbm, src_hbm, dst_hbm, ps_hbm, qd_hbm,
             src_v, dst_v, bufp, bufq, semp, semq):
    wid = lax.axis_index("s") * 2 + lax.axis_index("c")
    base = wid * EPW
    pltpu.sync_copy(src_hbm.at[pl.ds(base, EPW)], src_v)
    pltpu.sync_copy(dst_hbm.at[pl.ds(base, EPW)], dst_v)

    def chunk(c, carry):
        off = c * GB
        gp = pltpu.async_copy(p_hbm.at[src_v.at[pl.ds(off, GB)]], bufp, semp)
        gq = pltpu.async_copy(q_hbm.at[dst_v.at[pl.ds(off, GB)]], bufq, semq)
        gp.wait()
        gq.wait()
        wp = pltpu.async_copy(bufp, ps_hbm.at[pl.ds(base + off, GB)], semp)
        wq = pltpu.async_copy(bufq, qd_hbm.at[pl.ds(base + off, GB)], semq)
        wp.wait()
        wq.wait()
        return carry

    lax.fori_loop(0, NGC, chunk, 0)


def _sc_gather(P, Q, src, dst):
    f = functools.partial(
        pl.kernel,
        out_type=[
            jax.ShapeDtypeStruct((E, H), jnp.float32),
            jax.ShapeDtypeStruct((E, H), jnp.float32),
        ],
        mesh=plsc.VectorSubcoreMesh(core_axis_name="c", subcore_axis_name="s"),
        scratch_types=[
            pltpu.VMEM((EPW,), jnp.int32),
            pltpu.VMEM((EPW,), jnp.int32),
            pltpu.VMEM((GB, H), jnp.float32),
            pltpu.VMEM((GB, H), jnp.float32),
            pltpu.SemaphoreType.DMA,
            pltpu.SemaphoreType.DMA,
        ],
    )(_s1_body)
    return f(P, Q, src, dst)


def _copy16(src_ref, src_off, dst_ref, n16):
    for k in range(n16):
        dst_ref[pl.ds(k * 16, 16)] = src_ref[pl.ds(src_off + k * 16, 16)]


def _dyngather16(v, idx):
    dn = lax.GatherDimensionNumbers(
        offset_dims=(), collapsed_slice_dims=(0,), start_index_map=(0,))
    return lax.gather(v, idx[:, None], dn, slice_sizes=(1,),
                      mode=lax.GatherScatterMode.PROMISE_IN_BOUNDS)


def _s2_body(lg_hbm, dst_hbm, attn_hbm, lg_v, d20_v, ex_v, idxb, mxb_v,
             mxall_v, denb_v, lgc_v, dstc_v, attnc_v, zb_v, den_sh, max_sh):
    cid = lax.axis_index("c")
    sid = lax.axis_index("s")
    tbase = sid * 20000
    pltpu.sync_copy(lg_hbm.at[pl.ds(tbase, 20000)], lg_v)
    pltpu.sync_copy(dst_hbm.at[pl.ds(tbase, 20000)], d20_v)

    @pl.when(sid == 0)
    def _():
        def zz(i, c):
            zb_v[pl.ds(i * 16, 16)] = jnp.zeros((16,), jnp.float32)
            return c
        lax.fori_loop(0, 625, zz, 0)
        pltpu.sync_copy(zb_v, den_sh)

    mxb_v[...] = lg_v[pl.ds(0, 16)]
    def mx(i, c):
        mxb_v[...] = jnp.maximum(mxb_v[...], lg_v[pl.ds(i * 16, 16)])
        return c
    lax.fori_loop(1, 1250, mx, 0)
    pltpu.sync_copy(mxb_v, max_sh.at[pl.ds(sid * 16, 16)])
    plsc.subcore_barrier()

    pltpu.sync_copy(max_sh, mxall_v)
    gv = mxall_v[pl.ds(0, 16)]
    for k in range(1, 16):
        gv = jnp.maximum(gv, mxall_v[pl.ds(k * 16, 16)])
    it16 = lax.iota(jnp.int32, 16)
    for s in (1, 2, 4, 8):
        gv = jnp.maximum(gv, _dyngather16(gv, jnp.bitwise_xor(it16, s)))
    gmax = gv

    def exb(i, c):
        ex_v[pl.ds(i * 16, 16)] = jnp.exp(lg_v[pl.ds(i * 16, 16)] - gmax)
        return c
    lax.fori_loop(0, 1250, exb, 0)

    def sca(c, carry):
        off = c * 80
        _copy16(d20_v, off, idxb, 5)
        pltpu.sync_copy(ex_v.at[pl.ds(off, 80)], den_sh.at[idxb], add=True)
        return carry
    lax.fori_loop(0, 250, sca, 0)
    plsc.subcore_barrier()

    wbase = (sid * 2 + cid) * EPW
    pltpu.sync_copy(lg_hbm.at[pl.ds(wbase, EPW)], lgc_v)
    pltpu.sync_copy(dst_hbm.at[pl.ds(wbase, EPW)], dstc_v)

    def att(c, carry):
        off = c * 80
        _copy16(dstc_v, off, idxb, 5)
        pltpu.sync_copy(den_sh.at[idxb], denb_v)
        for k in range(5):
            sl = pl.ds(off + k * 16, 16)
            d16 = denb_v[pl.ds(k * 16, 16)]
            attnc_v[sl] = jnp.exp(lgc_v[sl] - gmax) / (d16 + 1e-16)
        return carry
    lax.fori_loop(0, NGC, att, 0)
    pltpu.sync_copy(attnc_v, attn_hbm.at[pl.ds(wbase, EPW)])


def _sc_softmax(logit, dst):
    f = functools.partial(
        pl.kernel,
        out_type=jax.ShapeDtypeStruct((E,), jnp.float32),
        mesh=plsc.VectorSubcoreMesh(core_axis_name="c", subcore_axis_name="s"),
        scratch_types=[
            pltpu.VMEM((20000,), jnp.float32),
            pltpu.VMEM((20000,), jnp.int32),
            pltpu.VMEM((20000,), jnp.float32),
            pltpu.VMEM((80,), jnp.int32),
            pltpu.VMEM((16,), jnp.float32),
            pltpu.VMEM((256,), jnp.float32),
            pltpu.VMEM((80,), jnp.float32),
            pltpu.VMEM((EPW,), jnp.float32),
            pltpu.VMEM((EPW,), jnp.int32),
            pltpu.VMEM((EPW,), jnp.float32),
            pltpu.VMEM((N,), jnp.float32),
            pltpu.VMEM_SHARED((N,), jnp.float32),
            pltpu.VMEM_SHARED((256,), jnp.float32),
        ],
    )(_s2_body)
    return f(logit, dst)


def _s3_body(m_hbm, dst_hbm, z_hbm, agg_hbm, dstc_v, idxb, mbuf, agg_sh):
    cid = lax.axis_index("c")
    sid = lax.axis_index("s")
    rows = pl.ds(sid * 624, 624)
    pltpu.sync_copy(z_hbm.at[rows], agg_sh.at[rows])

    @pl.when(sid == 0)
    def _():
        tail = pl.ds(9984, 16)
        pltpu.sync_copy(z_hbm.at[tail], agg_sh.at[tail])

    base = cid * (E // 2) + sid * EPW
    pltpu.sync_copy(dst_hbm.at[pl.ds(base, EPW)], dstc_v)
    plsc.subcore_barrier()

    def chunk(c, carry):
        off = c * 80
        _copy16(dstc_v, off, idxb, 5)
        pltpu.sync_copy(m_hbm.at[pl.ds(base + off, 80)], mbuf)
        pltpu.sync_copy(mbuf, agg_sh.at[idxb], add=True)
        return carry
    lax.fori_loop(0, NGC, chunk, 0)
    plsc.subcore_barrier()
    pltpu.sync_copy(agg_sh.at[rows], agg_hbm.at[cid].at[rows])

    @pl.when(sid == 0)
    def _():
        tail = pl.ds(9984, 16)
        pltpu.sync_copy(agg_sh.at[tail], agg_hbm.at[cid].at[tail])


def _sc_scatter(m, dst, zeros):
    f = functools.partial(
        pl.kernel,
        out_type=jax.ShapeDtypeStruct((2, N, H), jnp.float32),
        mesh=plsc.VectorSubcoreMesh(core_axis_name="c", subcore_axis_name="s"),
        scratch_types=[
            pltpu.VMEM((EPW,), jnp.int32),
            pltpu.VMEM((80,), jnp.int32),
            pltpu.VMEM((80, H), jnp.float32),
            pltpu.VMEM_SHARED((N, H), jnp.float32),
        ],
    )(_s3_body)
    return f(m, dst, zeros)


def _t2_body(e_ref, at_ref, m_ref):
    m_ref[...] = e_ref[...] * at_ref[...]


def _tc_scale(e, attn2d):
    return pl.pallas_call(
        _t2_body,
        grid=(NBE,),
        in_specs=[
            pl.BlockSpec((BE, H), lambda c: (c, 0)),
            pl.BlockSpec((BE, 1), lambda c: (c, 0)),
        ],
        out_specs=pl.BlockSpec((BE, H), lambda c: (c, 0)),
        out_shape=jax.ShapeDtypeStruct((E, H), jnp.float32),
    )(e, attn2d)


def _t0_body(nf_ref, ws_ref, wd_ref, p_ref, q_ref):
    nf = nf_ref[...]
    p_ref[...] = jnp.dot(nf, ws_ref[...], preferred_element_type=jnp.float32)
    q_ref[...] = jnp.dot(nf, wd_ref[...], preferred_element_type=jnp.float32)


def _tc_pq(nf, Ws, Wd):
    return pl.pallas_call(
        _t0_body,
        grid=(NBN,),
        in_specs=[
            pl.BlockSpec((BN, H), lambda c: (c, 0)),
            pl.BlockSpec((H, H), lambda c: (0, 0)),
            pl.BlockSpec((H, H), lambda c: (0, 0)),
        ],
        out_specs=[
            pl.BlockSpec((BN, H), lambda c: (c, 0)),
            pl.BlockSpec((BN, H), lambda c: (c, 0)),
        ],
        out_shape=[
            jax.ShapeDtypeStruct((N, H), jnp.float32),
            jax.ShapeDtypeStruct((N, H), jnp.float32),
        ],
    )(nf, Ws, Wd)


def _t1_body(ps_ref, qd_ref, ef_ref, we_ref, be_ref, wa_ref, e_ref, lg_ref):
    z = (ps_ref[...] + qd_ref[...]
         + jnp.dot(ef_ref[...], we_ref[...], preferred_element_type=jnp.float32)
         + be_ref[...])
    e = jnp.maximum(z, 0.0)
    e_ref[...] = e
    c = pl.program_id(0)
    lg_ref[pl.ds(c * BE, BE)] = jnp.sum(e * wa_ref[...], axis=1)


def _tc_edge(psrc, qdst, ef, We_ef, be, wa):
    ed = ef.shape[1]
    return pl.pallas_call(
        _t1_body,
        grid=(NBE,),
        in_specs=[
            pl.BlockSpec((BE, H), lambda c: (c, 0)),
            pl.BlockSpec((BE, H), lambda c: (c, 0)),
            pl.BlockSpec((BE, ed), lambda c: (c, 0)),
            pl.BlockSpec((ed, H), lambda c: (0, 0)),
            pl.BlockSpec((H,), lambda c: (0,)),
            pl.BlockSpec((H,), lambda c: (0,)),
        ],
        out_specs=[
            pl.BlockSpec((BE, H), lambda c: (c, 0)),
            pl.BlockSpec((E,), lambda c: (0,)),
        ],
        out_shape=[
            jax.ShapeDtypeStruct((E, H), jnp.float32),
            jax.ShapeDtypeStruct((E,), jnp.float32),
        ],
    )(psrc, qdst, ef, We_ef, be, wa)


def _t3_body(nf_ref, a0_ref, a1_ref, wna_ref, wnb_ref, bn_ref, ws_ref, wd_ref,
             n_ref, p_ref, q_ref):
    agg = a0_ref[...] + a1_ref[...]
    n_new = jnp.maximum(
        jnp.dot(nf_ref[...], wna_ref[...], preferred_element_type=jnp.float32)
        + jnp.dot(agg, wnb_ref[...], preferred_element_type=jnp.float32)
        + bn_ref[...], 0.0)
    n_ref[...] = n_new
    p_ref[...] = jnp.dot(n_new, ws_ref[...], preferred_element_type=jnp.float32)
    q_ref[...] = jnp.dot(n_new, wd_ref[...], preferred_element_type=jnp.float32)


def _tc_node(nf, agg0, agg1, Wn_a, Wn_b, bn, Ws_next, Wd_next):
    nd = nf.shape[1]
    return pl.pallas_call(
        _t3_body,
        grid=(NBN,),
        in_specs=[
            pl.BlockSpec((BN, nd), lambda c: (c, 0)),
            pl.BlockSpec((BN, H), lambda c: (c, 0)),
            pl.BlockSpec((BN, H), lambda c: (c, 0)),
            pl.BlockSpec((nd, H), lambda c: (0, 0)),
            pl.BlockSpec((H, H), lambda c: (0, 0)),
            pl.BlockSpec((H,), lambda c: (0,)),
            pl.BlockSpec((H, H), lambda c: (0, 0)),
            pl.BlockSpec((H, H), lambda c: (0, 0)),
        ],
        out_specs=[
            pl.BlockSpec((BN, H), lambda c: (c, 0)),
            pl.BlockSpec((BN, H), lambda c: (c, 0)),
            pl.BlockSpec((BN, H), lambda c: (c, 0)),
        ],
        out_shape=[
            jax.ShapeDtypeStruct((N, H), jnp.float32),
            jax.ShapeDtypeStruct((N, H), jnp.float32),
            jax.ShapeDtypeStruct((N, H), jnp.float32),
        ],
    )(nf, agg0, agg1, Wn_a, Wn_b, bn, Ws_next, Wd_next)


def _t3f_body(nf_ref, a0_ref, a1_ref, wna_ref, wnb_ref, bn_ref, wo_ref, bo_ref,
              u_ref):
    agg = a0_ref[...] + a1_ref[...]
    n_new = jnp.maximum(
        jnp.dot(nf_ref[...], wna_ref[...], preferred_element_type=jnp.float32)
        + jnp.dot(agg, wnb_ref[...], preferred_element_type=jnp.float32)
        + bn_ref[...], 0.0)
    u_ref[...] = (jnp.dot(n_new, wo_ref[...], preferred_element_type=jnp.float32)
                  + bo_ref[...])


def _tc_node_final(nf, agg0, agg1, Wn_a, Wn_b, bn, Wnode, bnode):
    nout = Wnode.shape[1]
    return pl.pallas_call(
        _t3f_body,
        grid=(NBN,),
        in_specs=[
            pl.BlockSpec((BN, H), lambda c: (c, 0)),
            pl.BlockSpec((BN, H), lambda c: (c, 0)),
            pl.BlockSpec((BN, H), lambda c: (c, 0)),
            pl.BlockSpec((H, H), lambda c: (0, 0)),
            pl.BlockSpec((H, H), lambda c: (0, 0)),
            pl.BlockSpec((H,), lambda c: (0,)),
            pl.BlockSpec((H, nout), lambda c: (0, 0)),
            pl.BlockSpec((nout,), lambda c: (0,)),
        ],
        out_specs=pl.BlockSpec((BN, nout), lambda c: (c, 0)),
        out_shape=jax.ShapeDtypeStruct((N, nout), jnp.float32),
    )(nf, agg0, agg1, Wn_a, Wn_b, bn, Wnode, bnode)


def _t4_body(ef_ref, w_ref, b_ref, u_ref):
    u_ref[...] = (jnp.dot(ef_ref[...], w_ref[...],
                          preferred_element_type=jnp.float32) + b_ref[...])


def _tc_edge_head(ef, Wedge, bedge):
    eout = Wedge.shape[1]
    return pl.pallas_call(
        _t4_body,
        grid=(NBE,),
        in_specs=[
            pl.BlockSpec((BE, H), lambda c: (c, 0)),
            pl.BlockSpec((H, eout), lambda c: (0, 0)),
            pl.BlockSpec((eout,), lambda c: (0,)),
        ],
        out_specs=pl.BlockSpec((BE, eout), lambda c: (c, 0)),
        out_shape=jax.ShapeDtypeStruct((E, eout), jnp.float32),
    )(ef, Wedge, bedge)


def _segment_softmax_agg(e, logit, dst, zeros):
    attn = _sc_softmax(logit, dst)
    m = _tc_scale(e, attn.reshape(E, 1))
    aggp = _sc_scatter(m, dst, zeros)
    return aggp[0], aggp[1]


def kernel(nf, ef, edge_index, We0, be0, wa0, Wn0, bn0, We1, be1, wa1, Wn1,
           bn1, We2, be2, wa2, Wn2, bn2, Wnode, bnode, Wedge, bedge):
    src = edge_index[0]
    dst = edge_index[1]
    params = [(We0, be0, wa0, Wn0, bn0), (We1, be1, wa1, Wn1, bn1),
              (We2, be2, wa2, Wn2, bn2)]
    nds = [nf.shape[1], H, H]
    splits = []
    for l, (We, be, wa, Wn, bn) in enumerate(params):
        nd = nds[l]
        We_s, We_d, We_e = We[:nd], We[nd:2 * nd], We[2 * nd:]
        Wn_a, Wn_b = Wn[:nd], Wn[nd:]
        splits.append((We_s, We_d, We_e, be, wa, Wn_a, Wn_b, bn))

    zeros = jnp.zeros((N, H), jnp.float32)
    P, Q = _tc_pq(nf, splits[0][0], splits[0][1])
    for l in range(3):
        We_s, We_d, We_e, be, wa, Wn_a, Wn_b, bn = splits[l]
        psrc, qdst = _sc_gather(P, Q, src, dst)
        e, logit = _tc_edge(psrc, qdst, ef, We_e, be, wa)
        agg0, agg1 = _segment_softmax_agg(e, logit, dst, zeros)
        if l < 2:
            P, Q = None, None
            nf, P, Q = _tc_node(nf, agg0, agg1, Wn_a, Wn_b, bn,
                                splits[l + 1][0], splits[l + 1][1])
        else:
            unf = _tc_node_final(nf, agg0, agg1, Wn_a, Wn_b, bn, Wnode, bnode)
        ef = e
    uef = _tc_edge_head(ef, Wedge, bedge)
    return unf, uef


# speedup vs baseline: 5.4865x; 1.0553x over previous
import functools
import jax
import jax.numpy as jnp
from jax import lax
from jax.experimental import pallas as pl
from jax.experimental.pallas import tpu as pltpu
from jax.experimental.pallas import tpu_sc as plsc

N = 10000
E = 320000
H = 128
NW = 32
EPW = E // NW
GB = 80
NGC = EPW // GB
BE = 2560
NBE = E // BE
BN = 1000
NBN = N // BN


_R = 5


def _s1_body(p_hbm, q_hbm, src_hbm, dst_hbm, ps_hbm, qd_hbm,
             src_v, dst_v, p0, p1, p2, p3, p4, q0, q1, q2, q3, q4,
             semg, semw):
    pbufs = [p0, p1, p2, p3, p4]
    qbufs = [q0, q1, q2, q3, q4]
    wid = lax.axis_index("s") * 2 + lax.axis_index("c")
    base = wid * EPW
    pltpu.sync_copy(src_hbm.at[pl.ds(base, EPW)], src_v)
    pltpu.sync_copy(dst_hbm.at[pl.ds(base, EPW)], dst_v)

    def group(g, carry):
        offs = [(g * _R + b) * GB for b in range(_R)]
        gs = []
        for b in range(_R):
            gs.append(pltpu.async_copy(
                p_hbm.at[src_v.at[pl.ds(offs[b], GB)]], pbufs[b], semg))
            gs.append(pltpu.async_copy(
                q_hbm.at[dst_v.at[pl.ds(offs[b], GB)]], qbufs[b], semw))
        for d in gs:
            d.wait()
        ws = []
        for b in range(_R):
            ws.append(pltpu.async_copy(
                pbufs[b], ps_hbm.at[pl.ds(base + offs[b], GB)], semg))
            ws.append(pltpu.async_copy(
                qbufs[b], qd_hbm.at[pl.ds(base + offs[b], GB)], semw))
        for d in ws:
            d.wait()
        return carry

    lax.fori_loop(0, NGC // _R, group, 0)


def _sc_gather(P, Q, src, dst):
    f = functools.partial(
        pl.kernel,
        out_type=[
            jax.ShapeDtypeStruct((E, H), jnp.float32),
            jax.ShapeDtypeStruct((E, H), jnp.float32),
        ],
        mesh=plsc.VectorSubcoreMesh(core_axis_name="c", subcore_axis_name="s"),
        scratch_types=[
            pltpu.VMEM((EPW,), jnp.int32),
            pltpu.VMEM((EPW,), jnp.int32),
        ] + [pltpu.VMEM((GB, H), jnp.float32)] * (2 * _R) + [
            pltpu.SemaphoreType.DMA,
            pltpu.SemaphoreType.DMA,
        ],
    )(_s1_body)
    return f(P, Q, src, dst)


def _copy16(src_ref, src_off, dst_ref, n16):
    for k in range(n16):
        dst_ref[pl.ds(k * 16, 16)] = src_ref[pl.ds(src_off + k * 16, 16)]


def _dyngather16(v, idx):
    dn = lax.GatherDimensionNumbers(
        offset_dims=(), collapsed_slice_dims=(0,), start_index_map=(0,))
    return lax.gather(v, idx[:, None], dn, slice_sizes=(1,),
                      mode=lax.GatherScatterMode.PROMISE_IN_BOUNDS)


def _s2_body(lg_hbm, dst_hbm, attn_hbm, lg_v, d20_v, ex_v,
             ib0, ib1, ib2, ib3, ib4, mxb_v, mxall_v,
             db0, db1, db2, db3, db4, lgc_v, dstc_v, attnc_v, zb_v,
             semx, semy, den_sh, max_sh):
    idxbs = [ib0, ib1, ib2, ib3, ib4]
    denbs = [db0, db1, db2, db3, db4]
    cid = lax.axis_index("c")
    sid = lax.axis_index("s")
    tbase = sid * 20000
    pltpu.sync_copy(lg_hbm.at[pl.ds(tbase, 20000)], lg_v)
    pltpu.sync_copy(dst_hbm.at[pl.ds(tbase, 20000)], d20_v)

    @pl.when(sid == 0)
    def _():
        def zz(i, c):
            zb_v[pl.ds(i * 16, 16)] = jnp.zeros((16,), jnp.float32)
            return c
        lax.fori_loop(0, 625, zz, 0)
        pltpu.sync_copy(zb_v, den_sh)

    mxb_v[...] = lg_v[pl.ds(0, 16)]
    def mx(i, c):
        mxb_v[...] = jnp.maximum(mxb_v[...], lg_v[pl.ds(i * 16, 16)])
        return c
    lax.fori_loop(1, 1250, mx, 0)
    pltpu.sync_copy(mxb_v, max_sh.at[pl.ds(sid * 16, 16)])
    plsc.subcore_barrier()

    pltpu.sync_copy(max_sh, mxall_v)
    gv = mxall_v[pl.ds(0, 16)]
    for k in range(1, 16):
        gv = jnp.maximum(gv, mxall_v[pl.ds(k * 16, 16)])
    it16 = lax.iota(jnp.int32, 16)
    for s in (1, 2, 4, 8):
        gv = jnp.maximum(gv, _dyngather16(gv, jnp.bitwise_xor(it16, s)))
    gmax = gv

    def exb(i, c):
        ex_v[pl.ds(i * 16, 16)] = jnp.exp(lg_v[pl.ds(i * 16, 16)] - gmax)
        return c
    lax.fori_loop(0, 1250, exb, 0)

    def sca(c, carry):
        off = c * 80
        _copy16(d20_v, off, idxbs[0], 5)
        pltpu.sync_copy(ex_v.at[pl.ds(off, 80)], den_sh.at[idxbs[0]],
                        add=True)
        return carry
    lax.fori_loop(0, 250, sca, 0)
    plsc.subcore_barrier()

    wbase = (sid * 2 + cid) * EPW
    pltpu.sync_copy(lg_hbm.at[pl.ds(wbase, EPW)], lgc_v)
    pltpu.sync_copy(dst_hbm.at[pl.ds(wbase, EPW)], dstc_v)

    def att(c, carry):
        off = c * 80
        _copy16(dstc_v, off, idxbs[0], 5)
        pltpu.sync_copy(den_sh.at[idxbs[0]], denbs[0])
        for k in range(5):
            sl = pl.ds(off + k * 16, 16)
            d16 = denbs[0][pl.ds(k * 16, 16)]
            attnc_v[sl] = jnp.exp(lgc_v[sl] - gmax) / (d16 + 1e-16)
        return carry
    lax.fori_loop(0, NGC, att, 0)
    pltpu.sync_copy(attnc_v, attn_hbm.at[pl.ds(wbase, EPW)])


def _sc_softmax(logit, dst):
    f = functools.partial(
        pl.kernel,
        out_type=jax.ShapeDtypeStruct((E,), jnp.float32),
        mesh=plsc.VectorSubcoreMesh(core_axis_name="c", subcore_axis_name="s"),
        scratch_types=[
            pltpu.VMEM((20000,), jnp.float32),
            pltpu.VMEM((20000,), jnp.int32),
            pltpu.VMEM((20000,), jnp.float32),
        ] + [pltpu.VMEM((80,), jnp.int32)] * _R + [
            pltpu.VMEM((16,), jnp.float32),
            pltpu.VMEM((256,), jnp.float32),
        ] + [pltpu.VMEM((80,), jnp.float32)] * _R + [
            pltpu.VMEM((EPW,), jnp.float32),
            pltpu.VMEM((EPW,), jnp.int32),
            pltpu.VMEM((EPW,), jnp.float32),
            pltpu.VMEM((N,), jnp.float32),
            pltpu.SemaphoreType.DMA,
            pltpu.SemaphoreType.DMA,
            pltpu.VMEM_SHARED((N,), jnp.float32),
            pltpu.VMEM_SHARED((256,), jnp.float32),
        ],
    )(_s2_body)
    return f(logit, dst)


HH = H // 2


def _s3_body(m_hbm, dst_hbm, z_hbm, agg_hbm, dstc_v,
             ib0, ib1, ib2, ib3, ib4, mb0, mb1, mb2, mb3, mb4,
             seml, sems, agg_sh):
    cid = lax.axis_index("c")
    sid = lax.axis_index("s")
    rows = pl.ds(sid * 624, 624)
    tail = pl.ds(9984, 16)
    pltpu.sync_copy(z_hbm.at[rows], agg_sh.at[rows])

    @pl.when(sid == 0)
    def _():
        pltpu.sync_copy(z_hbm.at[tail], agg_sh.at[tail])

    base = cid * (E // 2) + sid * EPW
    pltpu.sync_copy(dst_hbm.at[pl.ds(base, EPW)], dstc_v)
    plsc.subcore_barrier()

    mbufs = [mb0, mb1, mb2, mb3, mb4]
    idxbs = [ib0, ib1, ib2, ib3, ib4]

    def chunk(c, carry):
        off = c * GB
        _copy16(dstc_v, off, idxbs[0], 5)
        pltpu.sync_copy(m_hbm.at[pl.ds(base + off, GB)], mbufs[0])
        pltpu.sync_copy(mbufs[0], agg_sh.at[idxbs[0]], add=True)
        return carry
    lax.fori_loop(0, NGC, chunk, 0)
    plsc.subcore_barrier()
    pltpu.sync_copy(agg_sh.at[rows], agg_hbm.at[cid].at[rows])

    @pl.when(sid == 0)
    def _():
        pltpu.sync_copy(agg_sh.at[tail], agg_hbm.at[cid].at[tail])


def _sc_scatter(m, dst, zeros):
    f = functools.partial(
        pl.kernel,
        out_type=jax.ShapeDtypeStruct((2, N, H), jnp.float32),
        mesh=plsc.VectorSubcoreMesh(core_axis_name="c", subcore_axis_name="s"),
        scratch_types=[pltpu.VMEM((EPW,), jnp.int32)]
        + [pltpu.VMEM((GB,), jnp.int32)] * _R
        + [pltpu.VMEM((GB, H), jnp.float32)] * _R
        + [
            pltpu.SemaphoreType.DMA,
            pltpu.SemaphoreType.DMA,
            pltpu.VMEM_SHARED((N, H), jnp.float32),
        ],
    )(_s3_body)
    return f(m, dst, zeros)


def _t2_body(e_ref, at_ref, m_ref):
    m_ref[...] = e_ref[...] * at_ref[...]


def _tc_scale(e, attn2d):
    return pl.pallas_call(
        _t2_body,
        grid=(NBE,),
        in_specs=[
            pl.BlockSpec((BE, H), lambda c: (c, 0)),
            pl.BlockSpec((BE, 1), lambda c: (c, 0)),
        ],
        out_specs=pl.BlockSpec((BE, H), lambda c: (c, 0)),
        out_shape=jax.ShapeDtypeStruct((E, H), jnp.float32),
    )(e, attn2d)


def _t0_body(nf_ref, ws_ref, wd_ref, p_ref, q_ref):
    nf = nf_ref[...]
    p_ref[...] = jnp.dot(nf, ws_ref[...], preferred_element_type=jnp.float32)
    q_ref[...] = jnp.dot(nf, wd_ref[...], preferred_element_type=jnp.float32)


def _tc_pq(nf, Ws, Wd):
    return pl.pallas_call(
        _t0_body,
        grid=(NBN,),
        in_specs=[
            pl.BlockSpec((BN, H), lambda c: (c, 0)),
            pl.BlockSpec((H, H), lambda c: (0, 0)),
            pl.BlockSpec((H, H), lambda c: (0, 0)),
        ],
        out_specs=[
            pl.BlockSpec((BN, H), lambda c: (c, 0)),
            pl.BlockSpec((BN, H), lambda c: (c, 0)),
        ],
        out_shape=[
            jax.ShapeDtypeStruct((N, H), jnp.float32),
            jax.ShapeDtypeStruct((N, H), jnp.float32),
        ],
    )(nf, Ws, Wd)


def _t1_body(ps_ref, qd_ref, ef_ref, we_ref, be_ref, wa_ref, e_ref, lg_ref):
    z = (ps_ref[...] + qd_ref[...]
         + jnp.dot(ef_ref[...], we_ref[...], preferred_element_type=jnp.float32)
         + be_ref[...])
    e = jnp.maximum(z, 0.0)
    e_ref[...] = e
    c = pl.program_id(0)
    lg_ref[pl.ds(c * BE, BE)] = jnp.sum(e * wa_ref[...], axis=1)


def _tc_edge(psrc, qdst, ef, We_ef, be, wa):
    ed = ef.shape[1]
    return pl.pallas_call(
        _t1_body,
        grid=(NBE,),
        in_specs=[
            pl.BlockSpec((BE, H), lambda c: (c, 0)),
            pl.BlockSpec((BE, H), lambda c: (c, 0)),
            pl.BlockSpec((BE, ed), lambda c: (c, 0)),
            pl.BlockSpec((ed, H), lambda c: (0, 0)),
            pl.BlockSpec((H,), lambda c: (0,)),
            pl.BlockSpec((H,), lambda c: (0,)),
        ],
        out_specs=[
            pl.BlockSpec((BE, H), lambda c: (c, 0)),
            pl.BlockSpec((E,), lambda c: (0,)),
        ],
        out_shape=[
            jax.ShapeDtypeStruct((E, H), jnp.float32),
            jax.ShapeDtypeStruct((E,), jnp.float32),
        ],
    )(psrc, qdst, ef, We_ef, be, wa)


def _t3_body(nf_ref, agg_ref, wna_ref, wnb_ref, bn_ref,
             ws_ref, wd_ref, n_ref, p_ref, q_ref):
    agg = agg_ref[0] + agg_ref[1]
    n_new = jnp.maximum(
        jnp.dot(nf_ref[...], wna_ref[...], preferred_element_type=jnp.float32)
        + jnp.dot(agg, wnb_ref[...], preferred_element_type=jnp.float32)
        + bn_ref[...], 0.0)
    n_ref[...] = n_new
    p_ref[...] = jnp.dot(n_new, ws_ref[...], preferred_element_type=jnp.float32)
    q_ref[...] = jnp.dot(n_new, wd_ref[...], preferred_element_type=jnp.float32)


def _tc_node(nf, aggp, Wn_a, Wn_b, bn, Ws_next, Wd_next):
    nd = nf.shape[1]
    return pl.pallas_call(
        _t3_body,
        grid=(NBN,),
        in_specs=[
            pl.BlockSpec((BN, nd), lambda c: (c, 0)),
            pl.BlockSpec((2, BN, H), lambda c: (0, c, 0)),
            pl.BlockSpec((nd, H), lambda c: (0, 0)),
            pl.BlockSpec((H, H), lambda c: (0, 0)),
            pl.BlockSpec((H,), lambda c: (0,)),
            pl.BlockSpec((H, H), lambda c: (0, 0)),
            pl.BlockSpec((H, H), lambda c: (0, 0)),
        ],
        out_specs=[
            pl.BlockSpec((BN, H), lambda c: (c, 0)),
            pl.BlockSpec((BN, H), lambda c: (c, 0)),
            pl.BlockSpec((BN, H), lambda c: (c, 0)),
        ],
        out_shape=[
            jax.ShapeDtypeStruct((N, H), jnp.float32),
            jax.ShapeDtypeStruct((N, H), jnp.float32),
            jax.ShapeDtypeStruct((N, H), jnp.float32),
        ],
    )(nf, aggp, Wn_a, Wn_b, bn, Ws_next, Wd_next)


def _t3f_body(nf_ref, agg_ref, wna_ref, wnb_ref, bn_ref,
              wo_ref, bo_ref, u_ref):
    agg = agg_ref[0] + agg_ref[1]
    n_new = jnp.maximum(
        jnp.dot(nf_ref[...], wna_ref[...], preferred_element_type=jnp.float32)
        + jnp.dot(agg, wnb_ref[...], preferred_element_type=jnp.float32)
        + bn_ref[...], 0.0)
    u_ref[...] = (jnp.dot(n_new, wo_ref[...], preferred_element_type=jnp.float32)
                  + bo_ref[...])


def _tc_node_final(nf, aggp, Wn_a, Wn_b, bn, Wnode, bnode):
    nout = Wnode.shape[1]
    return pl.pallas_call(
        _t3f_body,
        grid=(NBN,),
        in_specs=[
            pl.BlockSpec((BN, H), lambda c: (c, 0)),
            pl.BlockSpec((2, BN, H), lambda c: (0, c, 0)),
            pl.BlockSpec((H, H), lambda c: (0, 0)),
            pl.BlockSpec((H, H), lambda c: (0, 0)),
            pl.BlockSpec((H,), lambda c: (0,)),
            pl.BlockSpec((H, nout), lambda c: (0, 0)),
            pl.BlockSpec((nout,), lambda c: (0,)),
        ],
        out_specs=pl.BlockSpec((BN, nout), lambda c: (c, 0)),
        out_shape=jax.ShapeDtypeStruct((N, nout), jnp.float32),
    )(nf, aggp, Wn_a, Wn_b, bn, Wnode, bnode)


def _t4_body(ef_ref, w_ref, b_ref, u_ref):
    u_ref[...] = (jnp.dot(ef_ref[...], w_ref[...],
                          preferred_element_type=jnp.float32) + b_ref[...])


def _tc_edge_head(ef, Wedge, bedge):
    eout = Wedge.shape[1]
    return pl.pallas_call(
        _t4_body,
        grid=(NBE,),
        in_specs=[
            pl.BlockSpec((BE, H), lambda c: (c, 0)),
            pl.BlockSpec((H, eout), lambda c: (0, 0)),
            pl.BlockSpec((eout,), lambda c: (0,)),
        ],
        out_specs=pl.BlockSpec((BE, eout), lambda c: (c, 0)),
        out_shape=jax.ShapeDtypeStruct((E, eout), jnp.float32),
    )(ef, Wedge, bedge)


def _segment_softmax_agg(e, logit, dst, zeros):
    attn = _sc_softmax(logit, dst)
    m = _tc_scale(e, attn.reshape(E, 1))
    return _sc_scatter(m, dst, zeros)


def kernel(nf, ef, edge_index, We0, be0, wa0, Wn0, bn0, We1, be1, wa1, Wn1,
           bn1, We2, be2, wa2, Wn2, bn2, Wnode, bnode, Wedge, bedge):
    src = edge_index[0]
    dst = edge_index[1]
    params = [(We0, be0, wa0, Wn0, bn0), (We1, be1, wa1, Wn1, bn1),
              (We2, be2, wa2, Wn2, bn2)]
    nds = [nf.shape[1], H, H]
    splits = []
    for l, (We, be, wa, Wn, bn) in enumerate(params):
        nd = nds[l]
        We_s, We_d, We_e = We[:nd], We[nd:2 * nd], We[2 * nd:]
        Wn_a, Wn_b = Wn[:nd], Wn[nd:]
        splits.append((We_s, We_d, We_e, be, wa, Wn_a, Wn_b, bn))

    zeros = jnp.zeros((N, H), jnp.float32)
    P, Q = _tc_pq(nf, splits[0][0], splits[0][1])
    for l in range(3):
        We_s, We_d, We_e, be, wa, Wn_a, Wn_b, bn = splits[l]
        psrc, qdst = _sc_gather(P, Q, src, dst)
        e, logit = _tc_edge(psrc, qdst, ef, We_e, be, wa)
        aggp = _segment_softmax_agg(e, logit, dst, zeros)
        if l < 2:
            P, Q = None, None
            nf, P, Q = _tc_node(nf, aggp, Wn_a, Wn_b, bn,
                                splits[l + 1][0], splits[l + 1][1])
        else:
            unf = _tc_node_final(nf, aggp, Wn_a, Wn_b, bn, Wnode, bnode)
        ef = e
    uef = _tc_edge_head(ef, Wedge, bedge)
    return unf, uef


# speedup vs baseline: 5.5399x; 1.0097x over previous
import functools
import jax
import jax.numpy as jnp
from jax import lax
from jax.experimental import pallas as pl
from jax.experimental.pallas import tpu as pltpu
from jax.experimental.pallas import tpu_sc as plsc

N = 10000
E = 320000
H = 128
NW = 32
EPW = E // NW
GB = 80
NGC = EPW // GB
BE = 2560
NBE = E // BE
BN = 1000
NBN = N // BN


_R = 5


def _s1_body(p_hbm, q_hbm, src_hbm, dst_hbm, ps_hbm, qd_hbm,
             src_v, dst_v, p0, p1, p2, p3, p4, q0, q1, q2, q3, q4,
             semg, semw):
    pbufs = [p0, p1, p2, p3, p4]
    qbufs = [q0, q1, q2, q3, q4]
    wid = lax.axis_index("s") * 2 + lax.axis_index("c")
    base = wid * EPW
    pltpu.sync_copy(src_hbm.at[pl.ds(base, EPW)], src_v)
    pltpu.sync_copy(dst_hbm.at[pl.ds(base, EPW)], dst_v)

    def group(g, carry):
        offs = [(g * _R + b) * GB for b in range(_R)]
        gs = []
        for b in range(_R):
            gs.append(pltpu.async_copy(
                p_hbm.at[src_v.at[pl.ds(offs[b], GB)]], pbufs[b], semg))
            gs.append(pltpu.async_copy(
                q_hbm.at[dst_v.at[pl.ds(offs[b], GB)]], qbufs[b], semw))
        for d in gs:
            d.wait()
        ws = []
        for b in range(_R):
            ws.append(pltpu.async_copy(
                pbufs[b], ps_hbm.at[pl.ds(base + offs[b], GB)], semg))
            ws.append(pltpu.async_copy(
                qbufs[b], qd_hbm.at[pl.ds(base + offs[b], GB)], semw))
        for d in ws:
            d.wait()
        return carry

    lax.fori_loop(0, NGC // _R, group, 0)


def _sc_gather(P, Q, src, dst):
    f = functools.partial(
        pl.kernel,
        out_type=[
            jax.ShapeDtypeStruct((E, H), jnp.float32),
            jax.ShapeDtypeStruct((E, H), jnp.float32),
        ],
        mesh=plsc.VectorSubcoreMesh(core_axis_name="c", subcore_axis_name="s"),
        scratch_types=[
            pltpu.VMEM((EPW,), jnp.int32),
            pltpu.VMEM((EPW,), jnp.int32),
        ] + [pltpu.VMEM((GB, H), jnp.float32)] * (2 * _R) + [
            pltpu.SemaphoreType.DMA,
            pltpu.SemaphoreType.DMA,
        ],
    )(_s1_body)
    return f(P, Q, src, dst)


def _copy16(src_ref, src_off, dst_ref, n16):
    for k in range(n16):
        dst_ref[pl.ds(k * 16, 16)] = src_ref[pl.ds(src_off + k * 16, 16)]


def _dyngather16(v, idx):
    dn = lax.GatherDimensionNumbers(
        offset_dims=(), collapsed_slice_dims=(0,), start_index_map=(0,))
    return lax.gather(v, idx[:, None], dn, slice_sizes=(1,),
                      mode=lax.GatherScatterMode.PROMISE_IN_BOUNDS)


def _s2_body(lg_hbm, dst_hbm, attn_hbm, lg_v, d20_v, ex_v,
             ib0, ib1, ib2, ib3, ib4, mxb_v, mxall_v,
             db0, db1, db2, db3, db4, lgc_v, dstc_v, attnc_v, zb_v,
             semx, semy, den_sh, max_sh):
    idxbs = [ib0, ib1, ib2, ib3, ib4]
    denbs = [db0, db1, db2, db3, db4]
    cid = lax.axis_index("c")
    sid = lax.axis_index("s")
    tbase = sid * 20000
    pltpu.sync_copy(lg_hbm.at[pl.ds(tbase, 20000)], lg_v)
    pltpu.sync_copy(dst_hbm.at[pl.ds(tbase, 20000)], d20_v)

    @pl.when(sid == 0)
    def _():
        def zz(i, c):
            zb_v[pl.ds(i * 16, 16)] = jnp.zeros((16,), jnp.float32)
            return c
        lax.fori_loop(0, 625, zz, 0)
        pltpu.sync_copy(zb_v, den_sh)

    mxb_v[...] = lg_v[pl.ds(0, 16)]
    def mx(i, c):
        mxb_v[...] = jnp.maximum(mxb_v[...], lg_v[pl.ds(i * 16, 16)])
        return c
    lax.fori_loop(1, 1250, mx, 0)
    pltpu.sync_copy(mxb_v, max_sh.at[pl.ds(sid * 16, 16)])
    plsc.subcore_barrier()

    pltpu.sync_copy(max_sh, mxall_v)
    gv = mxall_v[pl.ds(0, 16)]
    for k in range(1, 16):
        gv = jnp.maximum(gv, mxall_v[pl.ds(k * 16, 16)])
    it16 = lax.iota(jnp.int32, 16)
    for s in (1, 2, 4, 8):
        gv = jnp.maximum(gv, _dyngather16(gv, jnp.bitwise_xor(it16, s)))
    gmax = gv

    def exb(i, c):
        ex_v[pl.ds(i * 16, 16)] = jnp.exp(lg_v[pl.ds(i * 16, 16)] - gmax)
        return c
    lax.fori_loop(0, 1250, exb, 0)

    def sca(c, carry):
        off = c * 80
        _copy16(d20_v, off, idxbs[0], 5)
        pltpu.sync_copy(ex_v.at[pl.ds(off, 80)], den_sh.at[idxbs[0]],
                        add=True)
        return carry
    lax.fori_loop(0, 250, sca, 0)
    plsc.subcore_barrier()

    wbase = (sid * 2 + cid) * EPW
    pltpu.sync_copy(lg_hbm.at[pl.ds(wbase, EPW)], lgc_v)
    pltpu.sync_copy(dst_hbm.at[pl.ds(wbase, EPW)], dstc_v)

    def att(g, carry):
        offs = [(g * _R + b) * 80 for b in range(_R)]
        gds = []
        for b in range(_R):
            _copy16(dstc_v, offs[b], idxbs[b], 5)
            gds.append(pltpu.async_copy(den_sh.at[idxbs[b]], denbs[b], semy))
        for d in gds:
            d.wait()
        for b in range(_R):
            for k in range(5):
                sl = pl.ds(offs[b] + k * 16, 16)
                d16 = denbs[b][pl.ds(k * 16, 16)]
                attnc_v[sl] = jnp.exp(lgc_v[sl] - gmax) / (d16 + 1e-16)
        return carry
    lax.fori_loop(0, NGC // _R, att, 0)
    pltpu.sync_copy(attnc_v, attn_hbm.at[pl.ds(wbase, EPW)])


def _sc_softmax(logit, dst):
    f = functools.partial(
        pl.kernel,
        out_type=jax.ShapeDtypeStruct((E,), jnp.float32),
        mesh=plsc.VectorSubcoreMesh(core_axis_name="c", subcore_axis_name="s"),
        scratch_types=[
            pltpu.VMEM((20000,), jnp.float32),
            pltpu.VMEM((20000,), jnp.int32),
            pltpu.VMEM((20000,), jnp.float32),
        ] + [pltpu.VMEM((80,), jnp.int32)] * _R + [
            pltpu.VMEM((16,), jnp.float32),
            pltpu.VMEM((256,), jnp.float32),
        ] + [pltpu.VMEM((80,), jnp.float32)] * _R + [
            pltpu.VMEM((EPW,), jnp.float32),
            pltpu.VMEM((EPW,), jnp.int32),
            pltpu.VMEM((EPW,), jnp.float32),
            pltpu.VMEM((N,), jnp.float32),
            pltpu.SemaphoreType.DMA,
            pltpu.SemaphoreType.DMA,
            pltpu.VMEM_SHARED((N,), jnp.float32),
            pltpu.VMEM_SHARED((256,), jnp.float32),
        ],
    )(_s2_body)
    return f(logit, dst)


HH = H // 2


def _s3_body(m_hbm, dst_hbm, z_hbm, agg_hbm, dstc_v,
             ib0, ib1, ib2, ib3, ib4, mb0, mb1, mb2, mb3, mb4,
             seml, sems, agg_sh):
    cid = lax.axis_index("c")
    sid = lax.axis_index("s")
    rows = pl.ds(sid * 624, 624)
    tail = pl.ds(9984, 16)
    pltpu.sync_copy(z_hbm.at[rows], agg_sh.at[rows])

    @pl.when(sid == 0)
    def _():
        pltpu.sync_copy(z_hbm.at[tail], agg_sh.at[tail])

    base = cid * (E // 2) + sid * EPW
    pltpu.sync_copy(dst_hbm.at[pl.ds(base, EPW)], dstc_v)
    plsc.subcore_barrier()

    mbufs = [mb0, mb1, mb2, mb3, mb4]
    idxbs = [ib0, ib1, ib2, ib3, ib4]

    def chunk(c, carry):
        off = c * GB
        _copy16(dstc_v, off, idxbs[0], 5)
        pltpu.sync_copy(m_hbm.at[pl.ds(base + off, GB)], mbufs[0])
        pltpu.sync_copy(mbufs[0], agg_sh.at[idxbs[0]], add=True)
        return carry
    lax.fori_loop(0, NGC, chunk, 0)
    plsc.subcore_barrier()
    pltpu.sync_copy(agg_sh.at[rows], agg_hbm.at[cid].at[rows])

    @pl.when(sid == 0)
    def _():
        pltpu.sync_copy(agg_sh.at[tail], agg_hbm.at[cid].at[tail])


def _sc_scatter(m, dst, zeros):
    f = functools.partial(
        pl.kernel,
        out_type=jax.ShapeDtypeStruct((2, N, H), jnp.float32),
        mesh=plsc.VectorSubcoreMesh(core_axis_name="c", subcore_axis_name="s"),
        scratch_types=[pltpu.VMEM((EPW,), jnp.int32)]
        + [pltpu.VMEM((GB,), jnp.int32)] * _R
        + [pltpu.VMEM((GB, H), jnp.float32)] * _R
        + [
            pltpu.SemaphoreType.DMA,
            pltpu.SemaphoreType.DMA,
            pltpu.VMEM_SHARED((N, H), jnp.float32),
        ],
    )(_s3_body)
    return f(m, dst, zeros)


def _t2_body(e_ref, at_ref, m_ref):
    m_ref[...] = e_ref[...] * at_ref[...]


def _tc_scale(e, attn2d):
    return pl.pallas_call(
        _t2_body,
        grid=(NBE,),
        in_specs=[
            pl.BlockSpec((BE, H), lambda c: (c, 0)),
            pl.BlockSpec((BE, 1), lambda c: (c, 0)),
        ],
        out_specs=pl.BlockSpec((BE, H), lambda c: (c, 0)),
        out_shape=jax.ShapeDtypeStruct((E, H), jnp.float32),
    )(e, attn2d)


def _t0_body(nf_ref, ws_ref, wd_ref, p_ref, q_ref):
    nf = nf_ref[...]
    p_ref[...] = jnp.dot(nf, ws_ref[...], preferred_element_type=jnp.float32)
    q_ref[...] = jnp.dot(nf, wd_ref[...], preferred_element_type=jnp.float32)


def _tc_pq(nf, Ws, Wd):
    return pl.pallas_call(
        _t0_body,
        grid=(NBN,),
        in_specs=[
            pl.BlockSpec((BN, H), lambda c: (c, 0)),
            pl.BlockSpec((H, H), lambda c: (0, 0)),
            pl.BlockSpec((H, H), lambda c: (0, 0)),
        ],
        out_specs=[
            pl.BlockSpec((BN, H), lambda c: (c, 0)),
            pl.BlockSpec((BN, H), lambda c: (c, 0)),
        ],
        out_shape=[
            jax.ShapeDtypeStruct((N, H), jnp.float32),
            jax.ShapeDtypeStruct((N, H), jnp.float32),
        ],
    )(nf, Ws, Wd)


def _t1_body(ps_ref, qd_ref, ef_ref, we_ref, be_ref, wa_ref, e_ref, lg_ref):
    z = (ps_ref[...] + qd_ref[...]
         + jnp.dot(ef_ref[...], we_ref[...], preferred_element_type=jnp.float32)
         + be_ref[...])
    e = jnp.maximum(z, 0.0)
    e_ref[...] = e
    c = pl.program_id(0)
    lg_ref[pl.ds(c * BE, BE)] = jnp.sum(e * wa_ref[...], axis=1)


def _tc_edge(psrc, qdst, ef, We_ef, be, wa):
    ed = ef.shape[1]
    return pl.pallas_call(
        _t1_body,
        grid=(NBE,),
        in_specs=[
            pl.BlockSpec((BE, H), lambda c: (c, 0)),
            pl.BlockSpec((BE, H), lambda c: (c, 0)),
            pl.BlockSpec((BE, ed), lambda c: (c, 0)),
            pl.BlockSpec((ed, H), lambda c: (0, 0)),
            pl.BlockSpec((H,), lambda c: (0,)),
            pl.BlockSpec((H,), lambda c: (0,)),
        ],
        out_specs=[
            pl.BlockSpec((BE, H), lambda c: (c, 0)),
            pl.BlockSpec((E,), lambda c: (0,)),
        ],
        out_shape=[
            jax.ShapeDtypeStruct((E, H), jnp.float32),
            jax.ShapeDtypeStruct((E,), jnp.float32),
        ],
    )(psrc, qdst, ef, We_ef, be, wa)


def _t3_body(nf_ref, agg_ref, wna_ref, wnb_ref, bn_ref,
             ws_ref, wd_ref, n_ref, p_ref, q_ref):
    agg = agg_ref[0] + agg_ref[1]
    n_new = jnp.maximum(
        jnp.dot(nf_ref[...], wna_ref[...], preferred_element_type=jnp.float32)
        + jnp.dot(agg, wnb_ref[...], preferred_element_type=jnp.float32)
        + bn_ref[...], 0.0)
    n_ref[...] = n_new
    p_ref[...] = jnp.dot(n_new, ws_ref[...], preferred_element_type=jnp.float32)
    q_ref[...] = jnp.dot(n_new, wd_ref[...], preferred_element_type=jnp.float32)


def _tc_node(nf, aggp, Wn_a, Wn_b, bn, Ws_next, Wd_next):
    nd = nf.shape[1]
    return pl.pallas_call(
        _t3_body,
        grid=(NBN,),
        in_specs=[
            pl.BlockSpec((BN, nd), lambda c: (c, 0)),
            pl.BlockSpec((2, BN, H), lambda c: (0, c, 0)),
            pl.BlockSpec((nd, H), lambda c: (0, 0)),
            pl.BlockSpec((H, H), lambda c: (0, 0)),
            pl.BlockSpec((H,), lambda c: (0,)),
            pl.BlockSpec((H, H), lambda c: (0, 0)),
            pl.BlockSpec((H, H), lambda c: (0, 0)),
        ],
        out_specs=[
            pl.BlockSpec((BN, H), lambda c: (c, 0)),
            pl.BlockSpec((BN, H), lambda c: (c, 0)),
            pl.BlockSpec((BN, H), lambda c: (c, 0)),
        ],
        out_shape=[
            jax.ShapeDtypeStruct((N, H), jnp.float32),
            jax.ShapeDtypeStruct((N, H), jnp.float32),
            jax.ShapeDtypeStruct((N, H), jnp.float32),
        ],
    )(nf, aggp, Wn_a, Wn_b, bn, Ws_next, Wd_next)


def _t3f_body(nf_ref, agg_ref, wna_ref, wnb_ref, bn_ref,
              wo_ref, bo_ref, u_ref):
    agg = agg_ref[0] + agg_ref[1]
    n_new = jnp.maximum(
        jnp.dot(nf_ref[...], wna_ref[...], preferred_element_type=jnp.float32)
        + jnp.dot(agg, wnb_ref[...], preferred_element_type=jnp.float32)
        + bn_ref[...], 0.0)
    u_ref[...] = (jnp.dot(n_new, wo_ref[...], preferred_element_type=jnp.float32)
                  + bo_ref[...])


def _tc_node_final(nf, aggp, Wn_a, Wn_b, bn, Wnode, bnode):
    nout = Wnode.shape[1]
    return pl.pallas_call(
        _t3f_body,
        grid=(NBN,),
        in_specs=[
            pl.BlockSpec((BN, H), lambda c: (c, 0)),
            pl.BlockSpec((2, BN, H), lambda c: (0, c, 0)),
            pl.BlockSpec((H, H), lambda c: (0, 0)),
            pl.BlockSpec((H, H), lambda c: (0, 0)),
            pl.BlockSpec((H,), lambda c: (0,)),
            pl.BlockSpec((H, nout), lambda c: (0, 0)),
            pl.BlockSpec((nout,), lambda c: (0,)),
        ],
        out_specs=pl.BlockSpec((BN, nout), lambda c: (c, 0)),
        out_shape=jax.ShapeDtypeStruct((N, nout), jnp.float32),
    )(nf, aggp, Wn_a, Wn_b, bn, Wnode, bnode)


def _t4_body(ef_ref, w_ref, b_ref, u_ref):
    u_ref[...] = (jnp.dot(ef_ref[...], w_ref[...],
                          preferred_element_type=jnp.float32) + b_ref[...])


def _tc_edge_head(ef, Wedge, bedge):
    eout = Wedge.shape[1]
    return pl.pallas_call(
        _t4_body,
        grid=(NBE,),
        in_specs=[
            pl.BlockSpec((BE, H), lambda c: (c, 0)),
            pl.BlockSpec((H, eout), lambda c: (0, 0)),
            pl.BlockSpec((eout,), lambda c: (0,)),
        ],
        out_specs=pl.BlockSpec((BE, eout), lambda c: (c, 0)),
        out_shape=jax.ShapeDtypeStruct((E, eout), jnp.float32),
    )(ef, Wedge, bedge)


def _segment_softmax_agg(e, logit, dst, zeros):
    attn = _sc_softmax(logit, dst)
    m = _tc_scale(e, attn.reshape(E, 1))
    return _sc_scatter(m, dst, zeros)


def kernel(nf, ef, edge_index, We0, be0, wa0, Wn0, bn0, We1, be1, wa1, Wn1,
           bn1, We2, be2, wa2, Wn2, bn2, Wnode, bnode, Wedge, bedge):
    src = edge_index[0]
    dst = edge_index[1]
    params = [(We0, be0, wa0, Wn0, bn0), (We1, be1, wa1, Wn1, bn1),
              (We2, be2, wa2, Wn2, bn2)]
    nds = [nf.shape[1], H, H]
    splits = []
    for l, (We, be, wa, Wn, bn) in enumerate(params):
        nd = nds[l]
        We_s, We_d, We_e = We[:nd], We[nd:2 * nd], We[2 * nd:]
        Wn_a, Wn_b = Wn[:nd], Wn[nd:]
        splits.append((We_s, We_d, We_e, be, wa, Wn_a, Wn_b, bn))

    zeros = jnp.zeros((N, H), jnp.float32)
    P, Q = _tc_pq(nf, splits[0][0], splits[0][1])
    for l in range(3):
        We_s, We_d, We_e, be, wa, Wn_a, Wn_b, bn = splits[l]
        psrc, qdst = _sc_gather(P, Q, src, dst)
        e, logit = _tc_edge(psrc, qdst, ef, We_e, be, wa)
        aggp = _segment_softmax_agg(e, logit, dst, zeros)
        if l < 2:
            P, Q = None, None
            nf, P, Q = _tc_node(nf, aggp, Wn_a, Wn_b, bn,
                                splits[l + 1][0], splits[l + 1][1])
        else:
            unf = _tc_node_final(nf, aggp, Wn_a, Wn_b, bn, Wnode, bnode)
        ef = e
    uef = _tc_edge_head(ef, Wedge, bedge)
    return unf, uef


# speedup vs baseline: 5.6209x; 1.0146x over previous
import functools
import jax
import jax.numpy as jnp
from jax import lax
from jax.experimental import pallas as pl
from jax.experimental.pallas import tpu as pltpu
from jax.experimental.pallas import tpu_sc as plsc

N = 10000
E = 320000
H = 128
NW = 32
EPW = E // NW
GB = 80
NGC = EPW // GB
BE = 2560
NBE = E // BE
BN = 1000
NBN = N // BN


_R = 5


def _s1_body(p_hbm, q_hbm, src_hbm, dst_hbm, ps_hbm, qd_hbm,
             src_v, dst_v, p0, p1, p2, p3, p4, q0, q1, q2, q3, q4,
             semg, semw):
    pbufs = [p0, p1, p2, p3, p4]
    qbufs = [q0, q1, q2, q3, q4]
    wid = lax.axis_index("s") * 2 + lax.axis_index("c")
    base = wid * EPW
    pltpu.sync_copy(src_hbm.at[pl.ds(base, EPW)], src_v)
    pltpu.sync_copy(dst_hbm.at[pl.ds(base, EPW)], dst_v)

    def group(g, carry):
        offs = [(g * _R + b) * GB for b in range(_R)]
        gs = []
        for b in range(_R):
            gs.append(pltpu.async_copy(
                p_hbm.at[src_v.at[pl.ds(offs[b], GB)]], pbufs[b], semg))
            gs.append(pltpu.async_copy(
                q_hbm.at[dst_v.at[pl.ds(offs[b], GB)]], qbufs[b], semw))
        for d in gs:
            d.wait()
        ws = []
        for b in range(_R):
            ws.append(pltpu.async_copy(
                pbufs[b], ps_hbm.at[pl.ds(base + offs[b], GB)], semg))
            ws.append(pltpu.async_copy(
                qbufs[b], qd_hbm.at[pl.ds(base + offs[b], GB)], semw))
        for d in ws:
            d.wait()
        return carry

    lax.fori_loop(0, NGC // _R, group, 0)


def _sc_gather(P, Q, src, dst):
    f = functools.partial(
        pl.kernel,
        out_type=[
            jax.ShapeDtypeStruct((E, H), jnp.float32),
            jax.ShapeDtypeStruct((E, H), jnp.float32),
        ],
        mesh=plsc.VectorSubcoreMesh(core_axis_name="c", subcore_axis_name="s"),
        scratch_types=[
            pltpu.VMEM((EPW,), jnp.int32),
            pltpu.VMEM((EPW,), jnp.int32),
        ] + [pltpu.VMEM((GB, H), jnp.float32)] * (2 * _R) + [
            pltpu.SemaphoreType.DMA,
            pltpu.SemaphoreType.DMA,
        ],
    )(_s1_body)
    return f(P, Q, src, dst)


def _copy16(src_ref, src_off, dst_ref, n16):
    for k in range(n16):
        dst_ref[pl.ds(k * 16, 16)] = src_ref[pl.ds(src_off + k * 16, 16)]


def _dyngather16(v, idx):
    dn = lax.GatherDimensionNumbers(
        offset_dims=(), collapsed_slice_dims=(0,), start_index_map=(0,))
    return lax.gather(v, idx[:, None], dn, slice_sizes=(1,),
                      mode=lax.GatherScatterMode.PROMISE_IN_BOUNDS)


def _s2_body(lg_hbm, dst_hbm, attn_hbm, lg_v, d20_v, ex_v,
             ib0, ib1, ib2, ib3, ib4, mxb_v, mxall_v,
             db0, db1, db2, db3, db4, lgc_v, dstc_v, attnc_v, zb_v,
             semx, semy, den_sh, max_sh):
    idxbs = [ib0, ib1, ib2, ib3, ib4]
    denbs = [db0, db1, db2, db3, db4]
    cid = lax.axis_index("c")
    sid = lax.axis_index("s")
    tbase = sid * 20000
    pltpu.sync_copy(lg_hbm.at[pl.ds(tbase, 20000)], lg_v)
    pltpu.sync_copy(dst_hbm.at[pl.ds(tbase, 20000)], d20_v)

    @pl.when(sid == 0)
    def _():
        def zz(i, c):
            zb_v[pl.ds(i * 16, 16)] = jnp.zeros((16,), jnp.float32)
            return c
        lax.fori_loop(0, 625, zz, 0)
        pltpu.sync_copy(zb_v, den_sh)

    mxb_v[...] = lg_v[pl.ds(0, 16)]
    def mx(i, c):
        mxb_v[...] = jnp.maximum(mxb_v[...], lg_v[pl.ds(i * 16, 16)])
        return c
    lax.fori_loop(1, 1250, mx, 0)
    pltpu.sync_copy(mxb_v, max_sh.at[pl.ds(sid * 16, 16)])
    plsc.subcore_barrier()

    pltpu.sync_copy(max_sh, mxall_v)
    gv = mxall_v[pl.ds(0, 16)]
    for k in range(1, 16):
        gv = jnp.maximum(gv, mxall_v[pl.ds(k * 16, 16)])
    it16 = lax.iota(jnp.int32, 16)
    for s in (1, 2, 4, 8):
        gv = jnp.maximum(gv, _dyngather16(gv, jnp.bitwise_xor(it16, s)))
    gmax = gv

    def exb(i, c):
        ex_v[pl.ds(i * 16, 16)] = jnp.exp(lg_v[pl.ds(i * 16, 16)] - gmax)
        return c
    lax.fori_loop(0, 1250, exb, 0)

    def sca(g, carry):
        offs = [(g * _R + b) * 80 for b in range(_R)]
        scs = []
        for b in range(_R):
            _copy16(d20_v, offs[b], idxbs[b], 5)
            scs.append(pltpu.async_copy(
                ex_v.at[pl.ds(offs[b], 80)], den_sh.at[idxbs[b]], semx,
                add=True))
        for d in scs:
            d.wait()
        return carry
    lax.fori_loop(0, 250 // _R, sca, 0)
    plsc.subcore_barrier()

    wbase = (sid * 2 + cid) * EPW
    pltpu.sync_copy(lg_hbm.at[pl.ds(wbase, EPW)], lgc_v)
    pltpu.sync_copy(dst_hbm.at[pl.ds(wbase, EPW)], dstc_v)

    def att(g, carry):
        offs = [(g * _R + b) * 80 for b in range(_R)]
        gds = []
        for b in range(_R):
            _copy16(dstc_v, offs[b], idxbs[b], 5)
            gds.append(pltpu.async_copy(den_sh.at[idxbs[b]], denbs[b], semy))
        for d in gds:
            d.wait()
        for b in range(_R):
            for k in range(5):
                sl = pl.ds(offs[b] + k * 16, 16)
                d16 = denbs[b][pl.ds(k * 16, 16)]
                attnc_v[sl] = jnp.exp(lgc_v[sl] - gmax) / (d16 + 1e-16)
        return carry
    lax.fori_loop(0, NGC // _R, att, 0)
    pltpu.sync_copy(attnc_v, attn_hbm.at[pl.ds(wbase, EPW)])


def _sc_softmax(logit, dst):
    f = functools.partial(
        pl.kernel,
        out_type=jax.ShapeDtypeStruct((E,), jnp.float32),
        mesh=plsc.VectorSubcoreMesh(core_axis_name="c", subcore_axis_name="s"),
        scratch_types=[
            pltpu.VMEM((20000,), jnp.float32),
            pltpu.VMEM((20000,), jnp.int32),
            pltpu.VMEM((20000,), jnp.float32),
        ] + [pltpu.VMEM((80,), jnp.int32)] * _R + [
            pltpu.VMEM((16,), jnp.float32),
            pltpu.VMEM((256,), jnp.float32),
        ] + [pltpu.VMEM((80,), jnp.float32)] * _R + [
            pltpu.VMEM((EPW,), jnp.float32),
            pltpu.VMEM((EPW,), jnp.int32),
            pltpu.VMEM((EPW,), jnp.float32),
            pltpu.VMEM((N,), jnp.float32),
            pltpu.SemaphoreType.DMA,
            pltpu.SemaphoreType.DMA,
            pltpu.VMEM_SHARED((N,), jnp.float32),
            pltpu.VMEM_SHARED((256,), jnp.float32),
        ],
    )(_s2_body)
    return f(logit, dst)


HH = H // 2


def _s3_body(m_hbm, dst_hbm, z_hbm, agg_hbm, dstc_v,
             ib0, ib1, ib2, ib3, ib4, mb0, mb1, mb2, mb3, mb4,
             seml, sems, agg_sh):
    cid = lax.axis_index("c")
    sid = lax.axis_index("s")
    rows = pl.ds(sid * 624, 624)
    tail = pl.ds(9984, 16)
    pltpu.sync_copy(z_hbm.at[rows], agg_sh.at[rows])

    @pl.when(sid == 0)
    def _():
        pltpu.sync_copy(z_hbm.at[tail], agg_sh.at[tail])

    base = cid * (E // 2) + sid * EPW
    pltpu.sync_copy(dst_hbm.at[pl.ds(base, EPW)], dstc_v)
    plsc.subcore_barrier()

    mbufs = [mb0, mb1, mb2, mb3, mb4]
    idxbs = [ib0, ib1, ib2, ib3, ib4]

    def chunk(c, carry):
        off = c * GB
        _copy16(dstc_v, off, idxbs[0], 5)
        pltpu.sync_copy(m_hbm.at[pl.ds(base + off, GB)], mbufs[0])
        pltpu.sync_copy(mbufs[0], agg_sh.at[idxbs[0]], add=True)
        return carry
    lax.fori_loop(0, NGC, chunk, 0)
    plsc.subcore_barrier()
    pltpu.sync_copy(agg_sh.at[rows], agg_hbm.at[cid].at[rows])

    @pl.when(sid == 0)
    def _():
        pltpu.sync_copy(agg_sh.at[tail], agg_hbm.at[cid].at[tail])


def _sc_scatter(m, dst, zeros):
    f = functools.partial(
        pl.kernel,
        out_type=jax.ShapeDtypeStruct((2, N, H), jnp.float32),
        mesh=plsc.VectorSubcoreMesh(core_axis_name="c", subcore_axis_name="s"),
        scratch_types=[pltpu.VMEM((EPW,), jnp.int32)]
        + [pltpu.VMEM((GB,), jnp.int32)] * _R
        + [pltpu.VMEM((GB, H), jnp.float32)] * _R
        + [
            pltpu.SemaphoreType.DMA,
            pltpu.SemaphoreType.DMA,
            pltpu.VMEM_SHARED((N, H), jnp.float32),
        ],
    )(_s3_body)
    return f(m, dst, zeros)


def _t2_body(e_ref, at_ref, m_ref):
    m_ref[...] = e_ref[...] * at_ref[...]


def _tc_scale(e, attn2d):
    return pl.pallas_call(
        _t2_body,
        grid=(NBE,),
        in_specs=[
            pl.BlockSpec((BE, H), lambda c: (c, 0)),
            pl.BlockSpec((BE, 1), lambda c: (c, 0)),
        ],
        out_specs=pl.BlockSpec((BE, H), lambda c: (c, 0)),
        out_shape=jax.ShapeDtypeStruct((E, H), jnp.float32),
    )(e, attn2d)


def _t0_body(nf_ref, ws_ref, wd_ref, p_ref, q_ref):
    nf = nf_ref[...]
    p_ref[...] = jnp.dot(nf, ws_ref[...], preferred_element_type=jnp.float32)
    q_ref[...] = jnp.dot(nf, wd_ref[...], preferred_element_type=jnp.float32)


def _tc_pq(nf, Ws, Wd):
    return pl.pallas_call(
        _t0_body,
        grid=(NBN,),
        in_specs=[
            pl.BlockSpec((BN, H), lambda c: (c, 0)),
            pl.BlockSpec((H, H), lambda c: (0, 0)),
            pl.BlockSpec((H, H), lambda c: (0, 0)),
        ],
        out_specs=[
            pl.BlockSpec((BN, H), lambda c: (c, 0)),
            pl.BlockSpec((BN, H), lambda c: (c, 0)),
        ],
        out_shape=[
            jax.ShapeDtypeStruct((N, H), jnp.float32),
            jax.ShapeDtypeStruct((N, H), jnp.float32),
        ],
    )(nf, Ws, Wd)


def _t1_body(ps_ref, qd_ref, ef_ref, we_ref, be_ref, wa_ref, e_ref, lg_ref):
    z = (ps_ref[...] + qd_ref[...]
         + jnp.dot(ef_ref[...], we_ref[...], preferred_element_type=jnp.float32)
         + be_ref[...])
    e = jnp.maximum(z, 0.0)
    e_ref[...] = e
    c = pl.program_id(0)
    lg_ref[pl.ds(c * BE, BE)] = jnp.sum(e * wa_ref[...], axis=1)


def _tc_edge(psrc, qdst, ef, We_ef, be, wa):
    ed = ef.shape[1]
    return pl.pallas_call(
        _t1_body,
        grid=(NBE,),
        in_specs=[
            pl.BlockSpec((BE, H), lambda c: (c, 0)),
            pl.BlockSpec((BE, H), lambda c: (c, 0)),
            pl.BlockSpec((BE, ed), lambda c: (c, 0)),
            pl.BlockSpec((ed, H), lambda c: (0, 0)),
            pl.BlockSpec((H,), lambda c: (0,)),
            pl.BlockSpec((H,), lambda c: (0,)),
        ],
        out_specs=[
            pl.BlockSpec((BE, H), lambda c: (c, 0)),
            pl.BlockSpec((E,), lambda c: (0,)),
        ],
        out_shape=[
            jax.ShapeDtypeStruct((E, H), jnp.float32),
            jax.ShapeDtypeStruct((E,), jnp.float32),
        ],
    )(psrc, qdst, ef, We_ef, be, wa)


def _t3_body(nf_ref, agg_ref, wna_ref, wnb_ref, bn_ref,
             ws_ref, wd_ref, n_ref, p_ref, q_ref):
    agg = agg_ref[0] + agg_ref[1]
    n_new = jnp.maximum(
        jnp.dot(nf_ref[...], wna_ref[...], preferred_element_type=jnp.float32)
        + jnp.dot(agg, wnb_ref[...], preferred_element_type=jnp.float32)
        + bn_ref[...], 0.0)
    n_ref[...] = n_new
    p_ref[...] = jnp.dot(n_new, ws_ref[...], preferred_element_type=jnp.float32)
    q_ref[...] = jnp.dot(n_new, wd_ref[...], preferred_element_type=jnp.float32)


def _tc_node(nf, aggp, Wn_a, Wn_b, bn, Ws_next, Wd_next):
    nd = nf.shape[1]
    return pl.pallas_call(
        _t3_body,
        grid=(NBN,),
        in_specs=[
            pl.BlockSpec((BN, nd), lambda c: (c, 0)),
            pl.BlockSpec((2, BN, H), lambda c: (0, c, 0)),
            pl.BlockSpec((nd, H), lambda c: (0, 0)),
            pl.BlockSpec((H, H), lambda c: (0, 0)),
            pl.BlockSpec((H,), lambda c: (0,)),
            pl.BlockSpec((H, H), lambda c: (0, 0)),
            pl.BlockSpec((H, H), lambda c: (0, 0)),
        ],
        out_specs=[
            pl.BlockSpec((BN, H), lambda c: (c, 0)),
            pl.BlockSpec((BN, H), lambda c: (c, 0)),
            pl.BlockSpec((BN, H), lambda c: (c, 0)),
        ],
        out_shape=[
            jax.ShapeDtypeStruct((N, H), jnp.float32),
            jax.ShapeDtypeStruct((N, H), jnp.float32),
            jax.ShapeDtypeStruct((N, H), jnp.float32),
        ],
    )(nf, aggp, Wn_a, Wn_b, bn, Ws_next, Wd_next)


def _t3f_body(nf_ref, agg_ref, wna_ref, wnb_ref, bn_ref,
              wo_ref, bo_ref, u_ref):
    agg = agg_ref[0] + agg_ref[1]
    n_new = jnp.maximum(
        jnp.dot(nf_ref[...], wna_ref[...], preferred_element_type=jnp.float32)
        + jnp.dot(agg, wnb_ref[...], preferred_element_type=jnp.float32)
        + bn_ref[...], 0.0)
    u_ref[...] = (jnp.dot(n_new, wo_ref[...], preferred_element_type=jnp.float32)
                  + bo_ref[...])


def _tc_node_final(nf, aggp, Wn_a, Wn_b, bn, Wnode, bnode):
    nout = Wnode.shape[1]
    return pl.pallas_call(
        _t3f_body,
        grid=(NBN,),
        in_specs=[
            pl.BlockSpec((BN, H), lambda c: (c, 0)),
            pl.BlockSpec((2, BN, H), lambda c: (0, c, 0)),
            pl.BlockSpec((H, H), lambda c: (0, 0)),
            pl.BlockSpec((H, H), lambda c: (0, 0)),
            pl.BlockSpec((H,), lambda c: (0,)),
            pl.BlockSpec((H, nout), lambda c: (0, 0)),
            pl.BlockSpec((nout,), lambda c: (0,)),
        ],
        out_specs=pl.BlockSpec((BN, nout), lambda c: (c, 0)),
        out_shape=jax.ShapeDtypeStruct((N, nout), jnp.float32),
    )(nf, aggp, Wn_a, Wn_b, bn, Wnode, bnode)


def _t4_body(ef_ref, w_ref, b_ref, u_ref):
    u_ref[...] = (jnp.dot(ef_ref[...], w_ref[...],
                          preferred_element_type=jnp.float32) + b_ref[...])


def _tc_edge_head(ef, Wedge, bedge):
    eout = Wedge.shape[1]
    return pl.pallas_call(
        _t4_body,
        grid=(NBE,),
        in_specs=[
            pl.BlockSpec((BE, H), lambda c: (c, 0)),
            pl.BlockSpec((H, eout), lambda c: (0, 0)),
            pl.BlockSpec((eout,), lambda c: (0,)),
        ],
        out_specs=pl.BlockSpec((BE, eout), lambda c: (c, 0)),
        out_shape=jax.ShapeDtypeStruct((E, eout), jnp.float32),
    )(ef, Wedge, bedge)


def _segment_softmax_agg(e, logit, dst, zeros):
    attn = _sc_softmax(logit, dst)
    m = _tc_scale(e, attn.reshape(E, 1))
    return _sc_scatter(m, dst, zeros)


def kernel(nf, ef, edge_index, We0, be0, wa0, Wn0, bn0, We1, be1, wa1, Wn1,
           bn1, We2, be2, wa2, Wn2, bn2, Wnode, bnode, Wedge, bedge):
    src = edge_index[0]
    dst = edge_index[1]
    params = [(We0, be0, wa0, Wn0, bn0), (We1, be1, wa1, Wn1, bn1),
              (We2, be2, wa2, Wn2, bn2)]
    nds = [nf.shape[1], H, H]
    splits = []
    for l, (We, be, wa, Wn, bn) in enumerate(params):
        nd = nds[l]
        We_s, We_d, We_e = We[:nd], We[nd:2 * nd], We[2 * nd:]
        Wn_a, Wn_b = Wn[:nd], Wn[nd:]
        splits.append((We_s, We_d, We_e, be, wa, Wn_a, Wn_b, bn))

    zeros = jnp.zeros((N, H), jnp.float32)
    P, Q = _tc_pq(nf, splits[0][0], splits[0][1])
    for l in range(3):
        We_s, We_d, We_e, be, wa, Wn_a, Wn_b, bn = splits[l]
        psrc, qdst = _sc_gather(P, Q, src, dst)
        e, logit = _tc_edge(psrc, qdst, ef, We_e, be, wa)
        aggp = _segment_softmax_agg(e, logit, dst, zeros)
        if l < 2:
            P, Q = None, None
            nf, P, Q = _tc_node(nf, aggp, Wn_a, Wn_b, bn,
                                splits[l + 1][0], splits[l + 1][1])
        else:
            unf = _tc_node_final(nf, aggp, Wn_a, Wn_b, bn, Wnode, bnode)
        ef = e
    uef = _tc_edge_head(ef, Wedge, bedge)
    return unf, uef


# speedup vs baseline: 5.8347x; 1.0380x over previous
import functools
import jax
import jax.numpy as jnp
from jax import lax
from jax.experimental import pallas as pl
from jax.experimental.pallas import tpu as pltpu
from jax.experimental.pallas import tpu_sc as plsc

N = 10000
E = 320000
H = 128
NW = 32
EPW = E // NW
GB = 80
NGC = EPW // GB
BE = 2560
NBE = E // BE
BN = 1000
NBN = N // BN


_R = 5


def _s1_body(p_hbm, q_hbm, src_hbm, dst_hbm, ps_hbm, qd_hbm,
             src_v, dst_v, p0, p1, p2, p3, p4, q0, q1, q2, q3, q4,
             semg, semw):
    pbufs = [p0, p1, p2, p3, p4]
    qbufs = [q0, q1, q2, q3, q4]
    wid = lax.axis_index("s") * 2 + lax.axis_index("c")
    base = wid * EPW
    pltpu.sync_copy(src_hbm.at[pl.ds(base, EPW)], src_v)
    pltpu.sync_copy(dst_hbm.at[pl.ds(base, EPW)], dst_v)

    def group(g, carry):
        offs = [(g * _R + b) * GB for b in range(_R)]
        gs = []
        for b in range(_R):
            gs.append(pltpu.async_copy(
                p_hbm.at[src_v.at[pl.ds(offs[b], GB)]], pbufs[b], semg))
            gs.append(pltpu.async_copy(
                q_hbm.at[dst_v.at[pl.ds(offs[b], GB)]], qbufs[b], semw))
        for d in gs:
            d.wait()
        ws = []
        for b in range(_R):
            ws.append(pltpu.async_copy(
                pbufs[b], ps_hbm.at[pl.ds(base + offs[b], GB)], semg))
            ws.append(pltpu.async_copy(
                qbufs[b], qd_hbm.at[pl.ds(base + offs[b], GB)], semw))
        for d in ws:
            d.wait()
        return carry

    lax.fori_loop(0, NGC // _R, group, 0)


def _sc_gather(P, Q, src, dst):
    f = functools.partial(
        pl.kernel,
        out_type=[
            jax.ShapeDtypeStruct((E, H), jnp.float32),
            jax.ShapeDtypeStruct((E, H), jnp.float32),
        ],
        mesh=plsc.VectorSubcoreMesh(core_axis_name="c", subcore_axis_name="s"),
        scratch_types=[
            pltpu.VMEM((EPW,), jnp.int32),
            pltpu.VMEM((EPW,), jnp.int32),
        ] + [pltpu.VMEM((GB, H), jnp.float32)] * (2 * _R) + [
            pltpu.SemaphoreType.DMA,
            pltpu.SemaphoreType.DMA,
        ],
    )(_s1_body)
    return f(P, Q, src, dst)


def _copy16(src_ref, src_off, dst_ref, n16):
    for k in range(n16):
        dst_ref[pl.ds(k * 16, 16)] = src_ref[pl.ds(src_off + k * 16, 16)]


def _dyngather16(v, idx):
    dn = lax.GatherDimensionNumbers(
        offset_dims=(), collapsed_slice_dims=(0,), start_index_map=(0,))
    return lax.gather(v, idx[:, None], dn, slice_sizes=(1,),
                      mode=lax.GatherScatterMode.PROMISE_IN_BOUNDS)


def _s2_body(lg_hbm, dst_hbm, attn_hbm, lg_v, d20_v, ex_v,
             ib0, ib1, ib2, ib3, ib4, mxb_v, mxall_v,
             db0, db1, db2, db3, db4, lgc_v, dstc_v, attnc_v, zb_v,
             semx, semy, den_sh, max_sh):
    idxbs = [ib0, ib1, ib2, ib3, ib4]
    denbs = [db0, db1, db2, db3, db4]
    cid = lax.axis_index("c")
    sid = lax.axis_index("s")
    tbase = sid * 20000
    pltpu.sync_copy(lg_hbm.at[pl.ds(tbase, 20000)], lg_v)
    pltpu.sync_copy(dst_hbm.at[pl.ds(tbase, 20000)], d20_v)

    @pl.when(sid == 0)
    def _():
        def zz(i, c):
            zb_v[pl.ds(i * 16, 16)] = jnp.zeros((16,), jnp.float32)
            return c
        lax.fori_loop(0, 625, zz, 0)
        pltpu.sync_copy(zb_v, den_sh)

    mxb_v[...] = lg_v[pl.ds(0, 16)]
    def mx(i, c):
        mxb_v[...] = jnp.maximum(mxb_v[...], lg_v[pl.ds(i * 16, 16)])
        return c
    lax.fori_loop(1, 1250, mx, 0)
    pltpu.sync_copy(mxb_v, max_sh.at[pl.ds(sid * 16, 16)])
    plsc.subcore_barrier()

    pltpu.sync_copy(max_sh, mxall_v)
    gv = mxall_v[pl.ds(0, 16)]
    for k in range(1, 16):
        gv = jnp.maximum(gv, mxall_v[pl.ds(k * 16, 16)])
    it16 = lax.iota(jnp.int32, 16)
    for s in (1, 2, 4, 8):
        gv = jnp.maximum(gv, _dyngather16(gv, jnp.bitwise_xor(it16, s)))
    gmax = gv

    def exb(i, c):
        ex_v[pl.ds(i * 16, 16)] = jnp.exp(lg_v[pl.ds(i * 16, 16)] - gmax)
        return c
    lax.fori_loop(0, 1250, exb, 0)

    def sca(g, carry):
        offs = [(g * _R + b) * 80 for b in range(_R)]
        scs = []
        for b in range(_R):
            _copy16(d20_v, offs[b], idxbs[b], 5)
            scs.append(pltpu.async_copy(
                ex_v.at[pl.ds(offs[b], 80)], den_sh.at[idxbs[b]], semx,
                add=True))
        for d in scs:
            d.wait()
        return carry
    lax.fori_loop(0, 250 // _R, sca, 0)
    plsc.subcore_barrier()

    wbase = (sid * 2 + cid) * EPW
    pltpu.sync_copy(lg_hbm.at[pl.ds(wbase, EPW)], lgc_v)
    pltpu.sync_copy(dst_hbm.at[pl.ds(wbase, EPW)], dstc_v)

    def att(g, carry):
        offs = [(g * _R + b) * 80 for b in range(_R)]
        gds = []
        for b in range(_R):
            _copy16(dstc_v, offs[b], idxbs[b], 5)
            gds.append(pltpu.async_copy(den_sh.at[idxbs[b]], denbs[b], semy))
        for d in gds:
            d.wait()
        for b in range(_R):
            for k in range(5):
                sl = pl.ds(offs[b] + k * 16, 16)
                d16 = denbs[b][pl.ds(k * 16, 16)]
                attnc_v[sl] = jnp.exp(lgc_v[sl] - gmax) / (d16 + 1e-16)
        return carry
    lax.fori_loop(0, NGC // _R, att, 0)
    pltpu.sync_copy(attnc_v, attn_hbm.at[pl.ds(wbase, EPW)])


def _sc_softmax(logit, dst):
    f = functools.partial(
        pl.kernel,
        out_type=jax.ShapeDtypeStruct((E,), jnp.float32),
        mesh=plsc.VectorSubcoreMesh(core_axis_name="c", subcore_axis_name="s"),
        scratch_types=[
            pltpu.VMEM((20000,), jnp.float32),
            pltpu.VMEM((20000,), jnp.int32),
            pltpu.VMEM((20000,), jnp.float32),
        ] + [pltpu.VMEM((80,), jnp.int32)] * _R + [
            pltpu.VMEM((16,), jnp.float32),
            pltpu.VMEM((256,), jnp.float32),
        ] + [pltpu.VMEM((80,), jnp.float32)] * _R + [
            pltpu.VMEM((EPW,), jnp.float32),
            pltpu.VMEM((EPW,), jnp.int32),
            pltpu.VMEM((EPW,), jnp.float32),
            pltpu.VMEM((N,), jnp.float32),
            pltpu.SemaphoreType.DMA,
            pltpu.SemaphoreType.DMA,
            pltpu.VMEM_SHARED((N,), jnp.float32),
            pltpu.VMEM_SHARED((256,), jnp.float32),
        ],
    )(_s2_body)
    return f(logit, dst)


HH = H // 2


def _s3_body(m_hbm, dst_hbm, z_hbm, agg_hbm, dstc_v,
             ib0, ib1, ib2, ib3, ib4, mb0, mb1, mb2, mb3, mb4,
             seml, sems, agg_sh):
    cid = lax.axis_index("c")
    sid = lax.axis_index("s")
    rows = pl.ds(sid * 624, 624)
    tail = pl.ds(9984, 16)
    pltpu.sync_copy(z_hbm.at[rows], agg_sh.at[rows])

    @pl.when(sid == 0)
    def _():
        pltpu.sync_copy(z_hbm.at[tail], agg_sh.at[tail])

    base = cid * (E // 2) + sid * EPW
    pltpu.sync_copy(dst_hbm.at[pl.ds(base, EPW)], dstc_v)
    plsc.subcore_barrier()

    mbufs = [mb0, mb1, mb2, mb3, mb4]
    idxbs = [ib0, ib1, ib2, ib3, ib4]

    def group(g, carry):
        offs = [(g * 2 + b) * GB for b in range(2)]
        lds = []
        for b in range(2):
            _copy16(dstc_v, offs[b], idxbs[b], 5)
            lds.append(pltpu.async_copy(
                m_hbm.at[pl.ds(base + offs[b], GB)], mbufs[b], seml))
        scs = []
        for b in range(2):
            lds[b].wait()
            scs.append(pltpu.async_copy(
                mbufs[b], agg_sh.at[idxbs[b]], sems, add=True))
        for d in scs:
            d.wait()
        return carry
    lax.fori_loop(0, NGC // 2, group, 0)

    def chunk(c, carry):
        off = c * GB
        _copy16(dstc_v, off, idxbs[0], 5)
        pltpu.sync_copy(m_hbm.at[pl.ds(base + off, GB)], mbufs[0])
        pltpu.sync_copy(mbufs[0], agg_sh.at[idxbs[0]], add=True)
        return carry
    lax.fori_loop(2 * (NGC // 2), NGC, chunk, 0)
    plsc.subcore_barrier()
    pltpu.sync_copy(agg_sh.at[rows], agg_hbm.at[cid].at[rows])

    @pl.when(sid == 0)
    def _():
        pltpu.sync_copy(agg_sh.at[tail], agg_hbm.at[cid].at[tail])


def _sc_scatter(m, dst, zeros):
    f = functools.partial(
        pl.kernel,
        out_type=jax.ShapeDtypeStruct((2, N, H), jnp.float32),
        mesh=plsc.VectorSubcoreMesh(core_axis_name="c", subcore_axis_name="s"),
        scratch_types=[pltpu.VMEM((EPW,), jnp.int32)]
        + [pltpu.VMEM((GB,), jnp.int32)] * _R
        + [pltpu.VMEM((GB, H), jnp.float32)] * _R
        + [
            pltpu.SemaphoreType.DMA,
            pltpu.SemaphoreType.DMA,
            pltpu.VMEM_SHARED((N, H), jnp.float32),
        ],
    )(_s3_body)
    return f(m, dst, zeros)


def _t2_body(e_ref, at_ref, m_ref):
    m_ref[...] = e_ref[...] * at_ref[...]


def _tc_scale(e, attn2d):
    return pl.pallas_call(
        _t2_body,
        grid=(NBE,),
        in_specs=[
            pl.BlockSpec((BE, H), lambda c: (c, 0)),
            pl.BlockSpec((BE, 1), lambda c: (c, 0)),
        ],
        out_specs=pl.BlockSpec((BE, H), lambda c: (c, 0)),
        out_shape=jax.ShapeDtypeStruct((E, H), jnp.float32),
    )(e, attn2d)


def _t0_body(nf_ref, ws_ref, wd_ref, p_ref, q_ref):
    nf = nf_ref[...]
    p_ref[...] = jnp.dot(nf, ws_ref[...], preferred_element_type=jnp.float32)
    q_ref[...] = jnp.dot(nf, wd_ref[...], preferred_element_type=jnp.float32)


def _tc_pq(nf, Ws, Wd):
    return pl.pallas_call(
        _t0_body,
        grid=(NBN,),
        in_specs=[
            pl.BlockSpec((BN, H), lambda c: (c, 0)),
            pl.BlockSpec((H, H), lambda c: (0, 0)),
            pl.BlockSpec((H, H), lambda c: (0, 0)),
        ],
        out_specs=[
            pl.BlockSpec((BN, H), lambda c: (c, 0)),
            pl.BlockSpec((BN, H), lambda c: (c, 0)),
        ],
        out_shape=[
            jax.ShapeDtypeStruct((N, H), jnp.float32),
            jax.ShapeDtypeStruct((N, H), jnp.float32),
        ],
    )(nf, Ws, Wd)


def _t1_body(ps_ref, qd_ref, ef_ref, we_ref, be_ref, wa_ref, e_ref, lg_ref):
    z = (ps_ref[...] + qd_ref[...]
         + jnp.dot(ef_ref[...], we_ref[...], preferred_element_type=jnp.float32)
         + be_ref[...])
    e = jnp.maximum(z, 0.0)
    e_ref[...] = e
    c = pl.program_id(0)
    lg_ref[pl.ds(c * BE, BE)] = jnp.sum(e * wa_ref[...], axis=1)


def _tc_edge(psrc, qdst, ef, We_ef, be, wa):
    ed = ef.shape[1]
    return pl.pallas_call(
        _t1_body,
        grid=(NBE,),
        in_specs=[
            pl.BlockSpec((BE, H), lambda c: (c, 0)),
            pl.BlockSpec((BE, H), lambda c: (c, 0)),
            pl.BlockSpec((BE, ed), lambda c: (c, 0)),
            pl.BlockSpec((ed, H), lambda c: (0, 0)),
            pl.BlockSpec((H,), lambda c: (0,)),
            pl.BlockSpec((H,), lambda c: (0,)),
        ],
        out_specs=[
            pl.BlockSpec((BE, H), lambda c: (c, 0)),
            pl.BlockSpec((E,), lambda c: (0,)),
        ],
        out_shape=[
            jax.ShapeDtypeStruct((E, H), jnp.float32),
            jax.ShapeDtypeStruct((E,), jnp.float32),
        ],
    )(psrc, qdst, ef, We_ef, be, wa)


def _t3_body(nf_ref, agg_ref, wna_ref, wnb_ref, bn_ref,
             ws_ref, wd_ref, n_ref, p_ref, q_ref):
    agg = agg_ref[0] + agg_ref[1]
    n_new = jnp.maximum(
        jnp.dot(nf_ref[...], wna_ref[...], preferred_element_type=jnp.float32)
        + jnp.dot(agg, wnb_ref[...], preferred_element_type=jnp.float32)
        + bn_ref[...], 0.0)
    n_ref[...] = n_new
    p_ref[...] = jnp.dot(n_new, ws_ref[...], preferred_element_type=jnp.float32)
    q_ref[...] = jnp.dot(n_new, wd_ref[...], preferred_element_type=jnp.float32)


def _tc_node(nf, aggp, Wn_a, Wn_b, bn, Ws_next, Wd_next):
    nd = nf.shape[1]
    return pl.pallas_call(
        _t3_body,
        grid=(NBN,),
        in_specs=[
            pl.BlockSpec((BN, nd), lambda c: (c, 0)),
            pl.BlockSpec((2, BN, H), lambda c: (0, c, 0)),
            pl.BlockSpec((nd, H), lambda c: (0, 0)),
            pl.BlockSpec((H, H), lambda c: (0, 0)),
            pl.BlockSpec((H,), lambda c: (0,)),
            pl.BlockSpec((H, H), lambda c: (0, 0)),
            pl.BlockSpec((H, H), lambda c: (0, 0)),
        ],
        out_specs=[
            pl.BlockSpec((BN, H), lambda c: (c, 0)),
            pl.BlockSpec((BN, H), lambda c: (c, 0)),
            pl.BlockSpec((BN, H), lambda c: (c, 0)),
        ],
        out_shape=[
            jax.ShapeDtypeStruct((N, H), jnp.float32),
            jax.ShapeDtypeStruct((N, H), jnp.float32),
            jax.ShapeDtypeStruct((N, H), jnp.float32),
        ],
    )(nf, aggp, Wn_a, Wn_b, bn, Ws_next, Wd_next)


def _t3f_body(nf_ref, agg_ref, wna_ref, wnb_ref, bn_ref,
              wo_ref, bo_ref, u_ref):
    agg = agg_ref[0] + agg_ref[1]
    n_new = jnp.maximum(
        jnp.dot(nf_ref[...], wna_ref[...], preferred_element_type=jnp.float32)
        + jnp.dot(agg, wnb_ref[...], preferred_element_type=jnp.float32)
        + bn_ref[...], 0.0)
    u_ref[...] = (jnp.dot(n_new, wo_ref[...], preferred_element_type=jnp.float32)
                  + bo_ref[...])


def _tc_node_final(nf, aggp, Wn_a, Wn_b, bn, Wnode, bnode):
    nout = Wnode.shape[1]
    return pl.pallas_call(
        _t3f_body,
        grid=(NBN,),
        in_specs=[
            pl.BlockSpec((BN, H), lambda c: (c, 0)),
            pl.BlockSpec((2, BN, H), lambda c: (0, c, 0)),
            pl.BlockSpec((H, H), lambda c: (0, 0)),
            pl.BlockSpec((H, H), lambda c: (0, 0)),
            pl.BlockSpec((H,), lambda c: (0,)),
            pl.BlockSpec((H, nout), lambda c: (0, 0)),
            pl.BlockSpec((nout,), lambda c: (0,)),
        ],
        out_specs=pl.BlockSpec((BN, nout), lambda c: (c, 0)),
        out_shape=jax.ShapeDtypeStruct((N, nout), jnp.float32),
    )(nf, aggp, Wn_a, Wn_b, bn, Wnode, bnode)


def _t4_body(ef_ref, w_ref, b_ref, u_ref):
    u_ref[...] = (jnp.dot(ef_ref[...], w_ref[...],
                          preferred_element_type=jnp.float32) + b_ref[...])


def _tc_edge_head(ef, Wedge, bedge):
    eout = Wedge.shape[1]
    return pl.pallas_call(
        _t4_body,
        grid=(NBE,),
        in_specs=[
            pl.BlockSpec((BE, H), lambda c: (c, 0)),
            pl.BlockSpec((H, eout), lambda c: (0, 0)),
            pl.BlockSpec((eout,), lambda c: (0,)),
        ],
        out_specs=pl.BlockSpec((BE, eout), lambda c: (c, 0)),
        out_shape=jax.ShapeDtypeStruct((E, eout), jnp.float32),
    )(ef, Wedge, bedge)


def _segment_softmax_agg(e, logit, dst, zeros):
    attn = _sc_softmax(logit, dst)
    m = _tc_scale(e, attn.reshape(E, 1))
    return _sc_scatter(m, dst, zeros)


def kernel(nf, ef, edge_index, We0, be0, wa0, Wn0, bn0, We1, be1, wa1, Wn1,
           bn1, We2, be2, wa2, Wn2, bn2, Wnode, bnode, Wedge, bedge):
    src = edge_index[0]
    dst = edge_index[1]
    params = [(We0, be0, wa0, Wn0, bn0), (We1, be1, wa1, Wn1, bn1),
              (We2, be2, wa2, Wn2, bn2)]
    nds = [nf.shape[1], H, H]
    splits = []
    for l, (We, be, wa, Wn, bn) in enumerate(params):
        nd = nds[l]
        We_s, We_d, We_e = We[:nd], We[nd:2 * nd], We[2 * nd:]
        Wn_a, Wn_b = Wn[:nd], Wn[nd:]
        splits.append((We_s, We_d, We_e, be, wa, Wn_a, Wn_b, bn))

    zeros = jnp.zeros((N, H), jnp.float32)
    P, Q = _tc_pq(nf, splits[0][0], splits[0][1])
    for l in range(3):
        We_s, We_d, We_e, be, wa, Wn_a, Wn_b, bn = splits[l]
        psrc, qdst = _sc_gather(P, Q, src, dst)
        e, logit = _tc_edge(psrc, qdst, ef, We_e, be, wa)
        aggp = _segment_softmax_agg(e, logit, dst, zeros)
        if l < 2:
            P, Q = None, None
            nf, P, Q = _tc_node(nf, aggp, Wn_a, Wn_b, bn,
                                splits[l + 1][0], splits[l + 1][1])
        else:
            unf = _tc_node_final(nf, aggp, Wn_a, Wn_b, bn, Wnode, bnode)
        ef = e
    uef = _tc_edge_head(ef, Wedge, bedge)
    return unf, uef
